# Initial kernel scaffold; baseline (speedup 1.0000x reference)
import functools

import jax
import jax.numpy as jnp
from jax import lax
from jax.experimental import pallas as pl
from jax.experimental.pallas import tpu as pltpu
from jax.experimental.pallas import tpu_sc as plsc

N = 50000
D = 73
DP = 80
E = 800000
NEG = 0.2
BSH = 8
BN = 256
NB = (N + BN - 1) // BN
NBP = 256
NW = 32
EW = E // NW
CK = 128
NCK_W = (EW + CK - 1) // CK
ESRT = E + CK * NB
EPAD = ESRT + CK
TMAX = (NB + NW - 1) // NW

_mesh = plsc.VectorSubcoreMesh(core_axis_name="c", subcore_axis_name="s")


def _wid():
    return lax.axis_index("s") * 2 + lax.axis_index("c")


def _lin_body(xf_ref, xb_ref, wlf, blf, wrf, brf, wlb, blb, wrb, brb,
              xlf_ref, xrf_ref, xlb_ref, xrb_ref):
    xf = xf_ref[...]
    xb = xb_ref[...]
    xlf_ref[...] = jnp.dot(xf, wlf[...], preferred_element_type=jnp.float32) + blf[...]
    xrf_ref[...] = jnp.dot(xf, wrf[...], preferred_element_type=jnp.float32) + brf[...]
    xlb_ref[...] = jnp.dot(xb, wlb[...], preferred_element_type=jnp.float32) + blb[...]
    xrb_ref[...] = jnp.dot(xb, wrb[...], preferred_element_type=jnp.float32) + brb[...]


def _linear(xf, xb, wlf, blf, wrf, brf, wlb, blb, wrb, brb):
    blk = 2000
    grid = N // blk
    xspec = pl.BlockSpec((blk, D), lambda i: (i, 0))
    wspec = pl.BlockSpec((D, DP), lambda i: (0, 0))
    bspec = pl.BlockSpec((1, DP), lambda i: (0, 0))
    ospec = pl.BlockSpec((blk, DP), lambda i: (i, 0))
    return pl.pallas_call(
        _lin_body,
        grid=grid,
        in_specs=[xspec, xspec, wspec, bspec, wspec, bspec,
                  wspec, bspec, wspec, bspec],
        out_specs=[ospec, ospec, ospec, ospec],
        out_shape=[jax.ShapeDtypeStruct((N, DP), jnp.float32)] * 4,
    )(xf, xb, wlf, blf, wrf, brf, wlb, blb, wrb, brb)


def _hist_body(dst_hbm, hist_hbm, dbuf, hv):
    w = _wid()
    pltpu.sync_copy(dst_hbm.at[pl.ds(w * EW, EW)], dbuf)
    z = jnp.zeros((16,), jnp.int32)
    for k in range(NBP // 16):
        hv[pl.ds(16 * k, 16)] = z

    def body(e, _):
        b = dbuf[e] >> BSH
        hv[b] = hv[b] + 1
        return 0

    lax.fori_loop(0, EW, body, 0)
    pltpu.sync_copy(hv, hist_hbm.at[w])


_hist = pl.kernel(
    _hist_body,
    out_type=jax.ShapeDtypeStruct((NW, NBP), jnp.int32),
    mesh=_mesh,
    scratch_types=[
        pltpu.VMEM((EW,), jnp.int32),
        pltpu.VMEM((NBP,), jnp.int32),
    ],
)


def _scat_body(src_hbm, dst_hbm, hist_hbm, srcs_hbm, dsts_hbm,
               astart_hbm, cnt_hbm, hall, cur, astart_v, cnt_v,
               sbuf, dbuf, posb):
    w = _wid()
    pltpu.sync_copy(hist_hbm, hall)

    def bloop(b, a):
        def wsum(i, s):
            return s + hall[i, b]

        cs = lax.fori_loop(0, NW, wsum, 0)
        mine = lax.fori_loop(0, w, wsum, 0)
        cnt_v[b] = cs
        astart_v[b] = a
        cur[b] = a + mine
        return (a + cs + CK - 1) & (-CK)

    total = lax.fori_loop(0, NB, bloop, 0)
    astart_v[NB] = total

    base = w * EW

    def chunk(i, _):
        off = base + i * CK
        ce = jnp.minimum(CK, EW - i * CK)
        pltpu.sync_copy(src_hbm.at[pl.ds(off, CK)], sbuf)
        pltpu.sync_copy(dst_hbm.at[pl.ds(off, CK)], dbuf)

        def ebody(j, _):
            b = dbuf[j] >> BSH
            p = cur[b]
            cur[b] = p + 1
            posb[j] = p
            return 0

        lax.fori_loop(0, ce, ebody, 0)

        def tbody(j, _):
            posb[j] = ESRT + j
            return 0

        lax.fori_loop(ce, CK, tbody, 0)
        pltpu.sync_copy(sbuf, srcs_hbm.at[posb])
        pltpu.sync_copy(dbuf, dsts_hbm.at[posb])
        return 0

    lax.fori_loop(0, NCK_W, chunk, 0)

    @pl.when(w == 0)
    def _():
        pltpu.sync_copy(astart_v, astart_hbm)
        pltpu.sync_copy(cnt_v, cnt_hbm)


_scat = pl.kernel(
    _scat_body,
    out_type=(
        jax.ShapeDtypeStruct((EPAD,), jnp.int32),
        jax.ShapeDtypeStruct((EPAD,), jnp.int32),
        jax.ShapeDtypeStruct((NBP,), jnp.int32),
        jax.ShapeDtypeStruct((NBP,), jnp.int32),
    ),
    mesh=_mesh,
    scratch_types=[
        pltpu.VMEM((NW, NBP), jnp.int32),
        pltpu.VMEM((NBP,), jnp.int32),
        pltpu.VMEM((NBP,), jnp.int32),
        pltpu.VMEM((NBP,), jnp.int32),
        pltpu.VMEM((CK,), jnp.int32),
        pltpu.VMEM((CK,), jnp.int32),
        pltpu.VMEM((CK,), jnp.int32),
    ],
)


def _main_body(xl_hbm, xr_hbm, att_hbm, srcs_hbm, dsts_hbm, astart_hbm,
               cnt_hbm, o_hbm, ex_hbm, starts_v, cnt_v, attv, sidx, didx,
               xlrow, xrrow, acc, den, albuf, exbuf, cofb):
    w = _wid()
    pltpu.sync_copy(astart_hbm, starts_v)
    pltpu.sync_copy(cnt_hbm, cnt_v)
    pltpu.sync_copy(att_hbm, attv)
    attk = [attv[pl.ds(16 * k, 16)] for k in range(5)]
    zf = jnp.zeros((16,), jnp.float32)
    zi = jnp.zeros((16,), jnp.int32)

    def clamp_idx(ref, hi):
        for kk in range(CK // 16):
            v = ref[pl.ds(16 * kk, 16)]
            ref[pl.ds(16 * kk, 16)] = jnp.minimum(jnp.maximum(v, zi), hi)

    def bucket_body(t, _):
        b = w + NW * t

        @pl.when(b < NB)
        def _():
            start = starts_v[b]
            n_e = cnt_v[b]
            nck = (n_e + CK - 1) // CK
            nbase = b * BN

            def zacc(r, _):
                for k in range(5):
                    acc[r, pl.ds(16 * k, 16)] = zf
                return 0

            lax.fori_loop(0, BN, zacc, 0)
            for kk in range(BN // 16):
                den[pl.ds(16 * kk, 16)] = zf

            def p1c(i, _):
                off = start + i * CK
                ce = jnp.minimum(CK, n_e - i * CK)
                pltpu.sync_copy(srcs_hbm.at[pl.ds(off, CK)], sidx)
                pltpu.sync_copy(dsts_hbm.at[pl.ds(off, CK)], didx)
                clamp_idx(sidx, N - 1)
                clamp_idx(didx, N - 1)
                pltpu.sync_copy(xl_hbm.at[sidx], xlrow)
                pltpu.sync_copy(xr_hbm.at[didx], xrrow)

                def eb(j, _):
                    av = zf
                    for k in range(5):
                        s = (xlrow[j, pl.ds(16 * k, 16)]
                             + xrrow[j, pl.ds(16 * k, 16)])
                        lr = jnp.maximum(s, NEG * s)
                        av = av + attk[k] * lr
                    albuf[j] = jnp.sum(av)
                    return 0

                lax.fori_loop(0, ce, eb, 0)
                for kk in range(CK // 16):
                    exbuf[pl.ds(16 * kk, 16)] = jnp.exp(
                        albuf[pl.ds(16 * kk, 16)])

                def db(j, _):
                    dl = didx[j] - nbase
                    den[dl] = den[dl] + exbuf[j]
                    return 0

                lax.fori_loop(0, ce, db, 0)
                pltpu.sync_copy(exbuf, ex_hbm.at[pl.ds(off, CK)])
                return 0

            lax.fori_loop(0, nck, p1c, 0)

            def p2c(i, _):
                off = start + i * CK
                ce = jnp.minimum(CK, n_e - i * CK)
                pltpu.sync_copy(srcs_hbm.at[pl.ds(off, CK)], sidx)
                pltpu.sync_copy(dsts_hbm.at[pl.ds(off, CK)], didx)
                clamp_idx(sidx, N - 1)
                pltpu.sync_copy(ex_hbm.at[pl.ds(off, CK)], exbuf)
                pltpu.sync_copy(xl_hbm.at[sidx], xlrow)
                for kk in range(CK // 16):
                    dl = didx[pl.ds(16 * kk, 16)] - nbase
                    dl = jnp.minimum(jnp.maximum(dl, zi), BN - 1)
                    dv = plsc.load_gather(den, [dl])
                    cofb[pl.ds(16 * kk, 16)] = (
                        exbuf[pl.ds(16 * kk, 16)] / (dv + 1e-16))

                def eb2(j, _):
                    dl = didx[j] - nbase
                    c = cofb[j]
                    for k in range(5):
                        acc[dl, pl.ds(16 * k, 16)] = (
                            acc[dl, pl.ds(16 * k, 16)]
                            + c * xlrow[j, pl.ds(16 * k, 16)])
                    return 0

                lax.fori_loop(0, ce, eb2, 0)
                return 0

            lax.fori_loop(0, nck, p2c, 0)
            pltpu.sync_copy(acc, o_hbm.at[pl.ds(nbase, BN)])

        return 0

    lax.fori_loop(0, TMAX, bucket_body, 0)


_main = pl.kernel(
    _main_body,
    out_type=(
        jax.ShapeDtypeStruct((NB * BN, DP), jnp.float32),
        jax.ShapeDtypeStruct((EPAD,), jnp.float32),
    ),
    mesh=_mesh,
    scratch_types=[
        pltpu.VMEM((NBP,), jnp.int32),
        pltpu.VMEM((NBP,), jnp.int32),
        pltpu.VMEM((DP,), jnp.float32),
        pltpu.VMEM((CK,), jnp.int32),
        pltpu.VMEM((CK,), jnp.int32),
        pltpu.VMEM((CK, DP), jnp.float32),
        pltpu.VMEM((CK, DP), jnp.float32),
        pltpu.VMEM((BN, DP), jnp.float32),
        pltpu.VMEM((BN,), jnp.float32),
        pltpu.VMEM((CK,), jnp.float32),
        pltpu.VMEM((CK,), jnp.float32),
        pltpu.VMEM((CK,), jnp.float32),
    ],
)


def _conv_sc(xl, xr, att80, ei):
    srcp = jnp.pad(ei[0], (0, CK))
    dstp = jnp.pad(ei[1], (0, CK))
    hist = _hist(dstp)
    srcs, dsts, astart, cnt = _scat(srcp, dstp, hist)
    o, _ = _main(xl, xr, att80, srcs, dsts, astart, cnt)
    return o


def _fin_body(of_ref, ob_ref, xf_ref, xb_ref, bf, bb, out_ref):
    f = jnp.maximum(of_ref[:, :D] + bf[:, :D] + xf_ref[...], 0.0)
    g = jnp.maximum(ob_ref[:, :D] + bb[:, :D] + xb_ref[...], 0.0)
    out_ref[...] = jnp.concatenate([f, g], axis=-1)


def _final(of, ob, xf, xb, bf, bb):
    blk = 2000
    grid = N // blk
    ospec = pl.BlockSpec((blk, DP), lambda i: (i, 0))
    xspec = pl.BlockSpec((blk, D), lambda i: (i, 0))
    bspec = pl.BlockSpec((1, DP), lambda i: (0, 0))
    return pl.pallas_call(
        _fin_body,
        grid=grid,
        in_specs=[ospec, ospec, xspec, xspec, bspec, bspec],
        out_specs=pl.BlockSpec((blk, 2 * D), lambda i: (i, 0)),
        out_shape=jax.ShapeDtypeStruct((N, 2 * D), jnp.float32),
    )(of, ob, xf, xb, bf, bb)


def kernel(x_fwd, edge_index_fwd, x_bwd, edge_index_bwd,
           Wl_f, bl_f, Wr_f, br_f, att_f, bias_f,
           Wl_b, bl_b, Wr_b, br_b, att_b, bias_b):
    padw = lambda m: jnp.pad(m, ((0, 0), (0, DP - D)))
    padv = lambda v: jnp.pad(v, (0, DP - D)).reshape(1, DP)
    xlf, xrf, xlb, xrb = _linear(
        x_fwd, x_bwd,
        padw(Wl_f), padv(bl_f), padw(Wr_f), padv(br_f),
        padw(Wl_b), padv(bl_b), padw(Wr_b), padv(br_b))
    of = _conv_sc(xlf, xrf, jnp.pad(att_f, (0, DP - D)), edge_index_fwd)
    ob = _conv_sc(xlb, xrb, jnp.pad(att_b, (0, DP - D)), edge_index_bwd)
    return _final(of, ob, x_fwd, x_bwd, padv(bias_f), padv(bias_b))


# speedup vs baseline: 2.7941x; 2.7941x over previous
import jax
import jax.numpy as jnp
from jax import lax
from jax.experimental import pallas as pl
from jax.experimental.pallas import tpu as pltpu
from jax.experimental.pallas import tpu_sc as plsc

N = 50000
D = 73
DP = 80
E = 800000
NEG = 0.2
BSH = 8
BN = 256
NB = (N + BN - 1) // BN
NBP = 256
NW = 32
EW = E // NW
CK = 128
NCK_W = (EW + CK - 1) // CK
ESRT = E + CK * NB
EPAD = ESRT + CK
TMAX = (NB + NW - 1) // NW

_mesh = plsc.VectorSubcoreMesh(core_axis_name="c", subcore_axis_name="s")


def _wid():
    return lax.axis_index("s") * 2 + lax.axis_index("c")


def _sget(ref, i):
    return ref[pl.ds(i, 1)][0]


def _lin_body(xf_ref, xb_ref, wlf, blf, wrf, brf, wlb, blb, wrb, brb,
              xlf_ref, xrf_ref, xlb_ref, xrb_ref):
    xf = xf_ref[...]
    xb = xb_ref[...]
    xlf_ref[...] = jnp.dot(xf, wlf[...], preferred_element_type=jnp.float32) + blf[...]
    xrf_ref[...] = jnp.dot(xf, wrf[...], preferred_element_type=jnp.float32) + brf[...]
    xlb_ref[...] = jnp.dot(xb, wlb[...], preferred_element_type=jnp.float32) + blb[...]
    xrb_ref[...] = jnp.dot(xb, wrb[...], preferred_element_type=jnp.float32) + brb[...]


def _linear(xf, xb, wlf, blf, wrf, brf, wlb, blb, wrb, brb):
    blk = 2000
    grid = N // blk
    xspec = pl.BlockSpec((blk, D), lambda i: (i, 0))
    wspec = pl.BlockSpec((D, DP), lambda i: (0, 0))
    bspec = pl.BlockSpec((1, DP), lambda i: (0, 0))
    ospec = pl.BlockSpec((blk, DP), lambda i: (i, 0))
    return pl.pallas_call(
        _lin_body,
        grid=grid,
        in_specs=[xspec, xspec, wspec, bspec, wspec, bspec,
                  wspec, bspec, wspec, bspec],
        out_specs=[ospec, ospec, ospec, ospec],
        out_shape=[jax.ShapeDtypeStruct((N, DP), jnp.float32)] * 4,
    )(xf, xb, wlf, blf, wrf, brf, wlb, blb, wrb, brb)


def _hist_body(dst_hbm, hist_hbm, dbuf, hv):
    w = _wid()
    pltpu.sync_copy(dst_hbm.at[pl.ds(pl.multiple_of(w * EW, 8), EW)], dbuf)
    z = jnp.zeros((16,), jnp.int32)
    for k in range(NBP // 16):
        hv[pl.ds(16 * k, 16)] = z

    def body(e, _):
        b = _sget(dbuf, e) >> BSH
        hv[pl.ds(b, 1)] = hv[pl.ds(b, 1)] + 1
        return 0

    lax.fori_loop(0, EW, body, 0)
    pltpu.sync_copy(hv, hist_hbm.at[w])


_hist = pl.kernel(
    _hist_body,
    out_type=jax.ShapeDtypeStruct((NW, NBP), jnp.int32),
    mesh=_mesh,
    scratch_types=[
        pltpu.VMEM((EW,), jnp.int32),
        pltpu.VMEM((NBP,), jnp.int32),
    ],
)


def _scat_body(src_hbm, dst_hbm, hist_hbm, srcs_hbm, dsts_hbm,
               astart_hbm, cnt_hbm, hall, cur, astart_v, cnt_v,
               sbuf, dbuf, posb):
    w = _wid()
    pltpu.sync_copy(hist_hbm, hall)
    zi = jnp.zeros((16,), jnp.int32)

    for kk in range(NBP // 16):
        sl = pl.ds(16 * kk, 16)

        def ws(i, c):
            return c + hall[i, sl]

        cnt_v[sl] = lax.fori_loop(0, NW, ws, zi)
        cur[sl] = lax.fori_loop(0, w, ws, zi)

    def bloop(b, a):
        cs = _sget(cnt_v, b)
        astart_v[pl.ds(b, 1)] = jnp.reshape(a, (1,))
        cur[pl.ds(b, 1)] = cur[pl.ds(b, 1)] + a
        return (a + cs + CK - 1) & (-CK)

    total = lax.fori_loop(0, NB, bloop, 0)
    astart_v[pl.ds(NB, 1)] = jnp.reshape(total, (1,))

    base = w * EW

    def chunk(i, _):
        off = pl.multiple_of(base + i * CK, 8)
        ce = jnp.minimum(CK, EW - i * CK)
        pltpu.sync_copy(src_hbm.at[pl.ds(off, CK)], sbuf)
        pltpu.sync_copy(dst_hbm.at[pl.ds(off, CK)], dbuf)

        def ebody(j, _):
            b = _sget(dbuf, j) >> BSH
            p = cur[pl.ds(b, 1)]
            cur[pl.ds(b, 1)] = p + 1
            posb[pl.ds(j, 1)] = p
            return 0

        lax.fori_loop(0, ce, ebody, 0)

        def tbody(j, _):
            posb[pl.ds(j, 1)] = jnp.reshape(ESRT + j, (1,))
            return 0

        lax.fori_loop(ce, CK, tbody, 0)
        pltpu.sync_copy(sbuf, srcs_hbm.at[posb])
        pltpu.sync_copy(dbuf, dsts_hbm.at[posb])
        return 0

    lax.fori_loop(0, NCK_W, chunk, 0)

    @pl.when(w == 0)
    def _():
        pltpu.sync_copy(astart_v, astart_hbm)
        pltpu.sync_copy(cnt_v, cnt_hbm)


_scat = pl.kernel(
    _scat_body,
    out_type=(
        jax.ShapeDtypeStruct((EPAD,), jnp.int32),
        jax.ShapeDtypeStruct((EPAD,), jnp.int32),
        jax.ShapeDtypeStruct((NBP,), jnp.int32),
        jax.ShapeDtypeStruct((NBP,), jnp.int32),
    ),
    mesh=_mesh,
    scratch_types=[
        pltpu.VMEM((NW, NBP), jnp.int32),
        pltpu.VMEM((NBP,), jnp.int32),
        pltpu.VMEM((NBP,), jnp.int32),
        pltpu.VMEM((NBP,), jnp.int32),
        pltpu.VMEM((CK,), jnp.int32),
        pltpu.VMEM((CK,), jnp.int32),
        pltpu.VMEM((CK,), jnp.int32),
    ],
)


def _main_body(xl_hbm, xr_hbm, att_hbm, srcs_hbm, dsts_hbm, astart_hbm,
               cnt_hbm, o_hbm, ex_hbm, starts_v, cnt_v, attv, sidx, didx,
               xlrow, xrrow, acc, den, albuf, exbuf):
    w = _wid()
    pltpu.sync_copy(astart_hbm, starts_v)
    pltpu.sync_copy(cnt_hbm, cnt_v)
    pltpu.sync_copy(att_hbm, attv)
    attk = [attv[pl.ds(16 * k, 16)] for k in range(5)]
    zf = jnp.zeros((16,), jnp.float32)
    zi = jnp.zeros((16,), jnp.int32)

    def clamp_idx(ref, hi):
        for kk in range(CK // 16):
            v = ref[pl.ds(16 * kk, 16)]
            ref[pl.ds(16 * kk, 16)] = jnp.minimum(jnp.maximum(v, zi), hi)

    def bucket_body(t, _):
        b = w + NW * t

        @pl.when(b < NB)
        def _():
            start = _sget(starts_v, b)
            n_e = _sget(cnt_v, b)
            nck = (n_e + CK - 1) >> 7
            nbase = b * BN

            def zacc(r, _):
                row = acc.at[r]
                for k in range(5):
                    row[pl.ds(16 * k, 16)] = zf
                return 0

            lax.fori_loop(0, BN, zacc, 0)
            for kk in range(BN // 16):
                den[pl.ds(16 * kk, 16)] = zf

            def p1c(i, _):
                off = pl.multiple_of(start + i * CK, CK)
                ce = jnp.minimum(CK, n_e - i * CK)
                pltpu.sync_copy(srcs_hbm.at[pl.ds(off, CK)], sidx)
                pltpu.sync_copy(dsts_hbm.at[pl.ds(off, CK)], didx)
                clamp_idx(sidx, N - 1)
                clamp_idx(didx, N - 1)
                pltpu.sync_copy(xl_hbm.at[sidx], xlrow)
                pltpu.sync_copy(xr_hbm.at[didx], xrrow)

                def eb(j, _):
                    lrow = xlrow.at[j]
                    rrow = xrrow.at[j]
                    av = zf
                    for k in range(5):
                        s = (lrow[pl.ds(16 * k, 16)]
                             + rrow[pl.ds(16 * k, 16)])
                        lr = jnp.maximum(s, NEG * s)
                        av = av + attk[k] * lr
                    for sh in (8, 4, 2, 1):
                        av = av + av[jnp.arange(16) ^ sh]
                    albuf[pl.ds(j, 1)] = av[0:1]
                    return 0

                lax.fori_loop(0, ce, eb, 0)
                for kk in range(CK // 16):
                    exbuf[pl.ds(16 * kk, 16)] = jnp.exp(
                        albuf[pl.ds(16 * kk, 16)])

                def db(j, _):
                    dl = _sget(didx, j) - nbase
                    den[pl.ds(dl, 1)] = den[pl.ds(dl, 1)] + exbuf[pl.ds(j, 1)]
                    return 0

                lax.fori_loop(0, ce, db, 0)
                pltpu.sync_copy(exbuf, ex_hbm.at[pl.ds(off, CK)])
                return 0

            lax.fori_loop(0, nck, p1c, 0)

            def p2c(i, _):
                off = pl.multiple_of(start + i * CK, CK)
                ce = jnp.minimum(CK, n_e - i * CK)
                pltpu.sync_copy(srcs_hbm.at[pl.ds(off, CK)], sidx)
                pltpu.sync_copy(dsts_hbm.at[pl.ds(off, CK)], didx)
                clamp_idx(sidx, N - 1)
                pltpu.sync_copy(ex_hbm.at[pl.ds(off, CK)], exbuf)
                pltpu.sync_copy(xl_hbm.at[sidx], xlrow)

                def eb2(j, _):
                    dl = _sget(didx, j) - nbase
                    c1 = exbuf[pl.ds(j, 1)] / (den[pl.ds(dl, 1)] + 1e-16)
                    c = c1[0]
                    arow = acc.at[dl]
                    lrow = xlrow.at[j]
                    for k in range(5):
                        sl = pl.ds(16 * k, 16)
                        arow[sl] = arow[sl] + c * lrow[sl]
                    return 0

                lax.fori_loop(0, ce, eb2, 0)
                return 0

            lax.fori_loop(0, nck, p2c, 0)
            pltpu.sync_copy(acc, o_hbm.at[pl.ds(nbase, BN)])

        return 0

    lax.fori_loop(0, TMAX, bucket_body, 0)


_main = pl.kernel(
    _main_body,
    compiler_params=pltpu.CompilerParams(use_tc_tiling_on_sc=False),
    out_type=(
        jax.ShapeDtypeStruct((NB * BN, DP), jnp.float32),
        jax.ShapeDtypeStruct((EPAD,), jnp.float32),
    ),
    mesh=_mesh,
    scratch_types=[
        pltpu.VMEM((NBP,), jnp.int32),
        pltpu.VMEM((NBP,), jnp.int32),
        pltpu.VMEM((DP,), jnp.float32),
        pltpu.VMEM((CK,), jnp.int32),
        pltpu.VMEM((CK,), jnp.int32),
        pltpu.VMEM((CK, DP), jnp.float32),
        pltpu.VMEM((CK, DP), jnp.float32),
        pltpu.VMEM((BN, DP), jnp.float32),
        pltpu.VMEM((BN,), jnp.float32),
        pltpu.VMEM((CK,), jnp.float32),
        pltpu.VMEM((CK,), jnp.float32),
    ],
)


def _conv_sc(xl, xr, att80, ei):
    srcp = jnp.pad(ei[0], (0, CK))
    dstp = jnp.pad(ei[1], (0, CK))
    hist = _hist(dstp)
    srcs, dsts, astart, cnt = _scat(srcp, dstp, hist)
    o, _ = _main(xl, xr, att80, srcs, dsts, astart, cnt)
    return o


def _fin_body(of_ref, ob_ref, xf_ref, xb_ref, bf, bb, out_ref):
    f = jnp.maximum(of_ref[:, :D] + bf[:, :D] + xf_ref[...], 0.0)
    g = jnp.maximum(ob_ref[:, :D] + bb[:, :D] + xb_ref[...], 0.0)
    out_ref[...] = jnp.concatenate([f, g], axis=-1)


def _final(of, ob, xf, xb, bf, bb):
    blk = 2000
    grid = N // blk
    ospec = pl.BlockSpec((blk, DP), lambda i: (i, 0))
    xspec = pl.BlockSpec((blk, D), lambda i: (i, 0))
    bspec = pl.BlockSpec((1, DP), lambda i: (0, 0))
    return pl.pallas_call(
        _fin_body,
        grid=grid,
        in_specs=[ospec, ospec, xspec, xspec, bspec, bspec],
        out_specs=pl.BlockSpec((blk, 2 * D), lambda i: (i, 0)),
        out_shape=jax.ShapeDtypeStruct((N, 2 * D), jnp.float32),
    )(of, ob, xf, xb, bf, bb)


def kernel(x_fwd, edge_index_fwd, x_bwd, edge_index_bwd,
           Wl_f, bl_f, Wr_f, br_f, att_f, bias_f,
           Wl_b, bl_b, Wr_b, br_b, att_b, bias_b):
    padw = lambda m: jnp.pad(m, ((0, 0), (0, DP - D)))
    padv = lambda v: jnp.pad(v, (0, DP - D)).reshape(1, DP)
    xlf, xrf, xlb, xrb = _linear(
        x_fwd, x_bwd,
        padw(Wl_f), padv(bl_f), padw(Wr_f), padv(br_f),
        padw(Wl_b), padv(bl_b), padw(Wr_b), padv(br_b))
    of = _conv_sc(xlf, xrf, jnp.pad(att_f, (0, DP - D)), edge_index_fwd)
    ob = _conv_sc(xlb, xrb, jnp.pad(att_b, (0, DP - D)), edge_index_bwd)
    return _final(of, ob, x_fwd, x_bwd, padv(bias_f), padv(bias_b))


# speedup vs baseline: 3.4172x; 1.2230x over previous
import jax
import jax.numpy as jnp
from jax import lax
from jax.experimental import pallas as pl
from jax.experimental.pallas import tpu as pltpu
from jax.experimental.pallas import tpu_sc as plsc

N = 50000
D = 73
DP = 80
E = 800000
NEG = 0.2
BSH = 8
BN = 256
NB = (N + BN - 1) // BN
NBP = 256
NW = 32
EW = E // NW
CK = 128
NCK_W = (EW + CK - 1) // CK
ESRT = E + CK * NB
EPAD = ESRT + CK
TMAX = (NB + NW - 1) // NW

_mesh = plsc.VectorSubcoreMesh(core_axis_name="c", subcore_axis_name="s")


def _wid():
    return lax.axis_index("s") * 2 + lax.axis_index("c")


def _sget(ref, i):
    return ref[pl.ds(i, 1)][0]


def _lin_body(xf_ref, xb_ref, wlf, blf, wrf, brf, wlb, blb, wrb, brb,
              xlf_ref, xrf_ref, xlb_ref, xrb_ref):
    xf = xf_ref[...]
    xb = xb_ref[...]
    xlf_ref[...] = jnp.dot(xf, wlf[...], preferred_element_type=jnp.float32) + blf[...]
    xrf_ref[...] = jnp.dot(xf, wrf[...], preferred_element_type=jnp.float32) + brf[...]
    xlb_ref[...] = jnp.dot(xb, wlb[...], preferred_element_type=jnp.float32) + blb[...]
    xrb_ref[...] = jnp.dot(xb, wrb[...], preferred_element_type=jnp.float32) + brb[...]


def _linear(xf, xb, wlf, blf, wrf, brf, wlb, blb, wrb, brb):
    blk = 2000
    grid = N // blk
    xspec = pl.BlockSpec((blk, D), lambda i: (i, 0))
    wspec = pl.BlockSpec((D, DP), lambda i: (0, 0))
    bspec = pl.BlockSpec((1, DP), lambda i: (0, 0))
    ospec = pl.BlockSpec((blk, DP), lambda i: (i, 0))
    return pl.pallas_call(
        _lin_body,
        grid=grid,
        in_specs=[xspec, xspec, wspec, bspec, wspec, bspec,
                  wspec, bspec, wspec, bspec],
        out_specs=[ospec, ospec, ospec, ospec],
        out_shape=[jax.ShapeDtypeStruct((N, DP), jnp.float32)] * 4,
    )(xf, xb, wlf, blf, wrf, brf, wlb, blb, wrb, brb)


def _hist_body(dst_hbm, hist_hbm, dbuf, hv, h1, h2, h3):
    w = _wid()
    pltpu.sync_copy(dst_hbm.at[pl.ds(pl.multiple_of(w * EW, 8), EW)], dbuf)
    z = jnp.zeros((16,), jnp.int32)
    for k in range(NBP // 16):
        hv[pl.ds(16 * k, 16)] = z
        h1[pl.ds(16 * k, 16)] = z
        h2[pl.ds(16 * k, 16)] = z
        h3[pl.ds(16 * k, 16)] = z
    hs = (hv, h1, h2, h3)

    def body(e, _):
        for q in range(4):
            b = _sget(dbuf, 4 * e + q) >> BSH
            hq = hs[q]
            hq[pl.ds(b, 1)] = hq[pl.ds(b, 1)] + 1
        return 0

    lax.fori_loop(0, EW // 4, body, 0)
    for k in range(NBP // 16):
        sl = pl.ds(16 * k, 16)
        hv[sl] = hv[sl] + h1[sl] + h2[sl] + h3[sl]
    pltpu.sync_copy(hv, hist_hbm.at[w])


_hist = pl.kernel(
    _hist_body,
    out_type=jax.ShapeDtypeStruct((NW, NBP), jnp.int32),
    mesh=_mesh,
    name="sc_hist",
    scratch_types=[
        pltpu.VMEM((EW,), jnp.int32),
        pltpu.VMEM((NBP,), jnp.int32),
        pltpu.VMEM((NBP,), jnp.int32),
        pltpu.VMEM((NBP,), jnp.int32),
        pltpu.VMEM((NBP,), jnp.int32),
    ],
)


def _scat_body(src_hbm, dst_hbm, hist_hbm, srcs_hbm, dsts_hbm,
               astart_hbm, cnt_hbm, hall, cur, astart_v, cnt_v,
               sbuf, dbuf, posb, sbuf1, dbuf1, posb1, sbuf2, dbuf2, posb2,
               semr, semw, semr1, semw1, semr2, semw2):
    w = _wid()
    pltpu.sync_copy(hist_hbm, hall)
    zi = jnp.zeros((16,), jnp.int32)

    for kk in range(NBP // 16):
        sl = pl.ds(16 * kk, 16)

        def ws(i, c):
            return c + hall[i, sl]

        cnt_v[sl] = lax.fori_loop(0, NW, ws, zi)
        cur[sl] = lax.fori_loop(0, w, ws, zi)

    def bloop(b, a):
        cs = _sget(cnt_v, b)
        astart_v[pl.ds(b, 1)] = jnp.reshape(a, (1,))
        cur[pl.ds(b, 1)] = cur[pl.ds(b, 1)] + a
        return (a + cs + CK - 1) & (-CK)

    total = lax.fori_loop(0, NB, bloop, 0)
    astart_v[pl.ds(NB, 1)] = jnp.reshape(total, (1,))

    base = w * EW
    bufs = ((sbuf, dbuf, posb, semr, semw),
            (sbuf1, dbuf1, posb1, semr1, semw1),
            (sbuf2, dbuf2, posb2, semr2, semw2))

    def issue_read(p, i):
        off = pl.multiple_of(base + i * CK, 8)
        s = bufs[p]
        pltpu.async_copy(src_hbm.at[pl.ds(off, CK)], s[0], s[3])
        pltpu.async_copy(dst_hbm.at[pl.ds(off, CK)], s[1], s[3])

    def wait_read(p):
        s = bufs[p]
        pltpu.make_async_copy(src_hbm.at[pl.ds(0, CK)], s[0], s[3]).wait()
        pltpu.make_async_copy(dst_hbm.at[pl.ds(0, CK)], s[1], s[3]).wait()

    def issue_scat(p):
        s = bufs[p]
        pltpu.async_copy(s[0], srcs_hbm.at[s[2]], s[4])
        pltpu.async_copy(s[1], dsts_hbm.at[s[2]], s[4])

    def wait_scat(p):
        s = bufs[p]
        pltpu.make_async_copy(s[0], srcs_hbm.at[s[2]], s[4]).wait()
        pltpu.make_async_copy(s[1], dsts_hbm.at[s[2]], s[4]).wait()

    issue_read(0, 0)

    def chunk3(i3, _):
        for sub in range(3):
            i = i3 * 3 + sub
            par = sub
            nxt = (sub + 1) % 3

            @pl.when(i < NCK_W)
            def _():
                @pl.when(i + 1 < NCK_W)
                def _():
                    @pl.when(i >= 2)
                    def _():
                        wait_scat(nxt)
                    issue_read(nxt, i + 1)
                wait_read(par)
                s = bufs[par]
                sdbuf, sposb = s[1], s[2]
                ce = jnp.minimum(CK, EW - i * CK)

                def ebody(j, _):
                    b = _sget(sdbuf, j) >> BSH
                    p = cur[pl.ds(b, 1)]
                    cur[pl.ds(b, 1)] = p + 1
                    sposb[pl.ds(j, 1)] = p
                    return 0

                lax.fori_loop(0, ce, ebody, 0)

                def tbody(j, _):
                    sposb[pl.ds(j, 1)] = jnp.reshape(ESRT + j, (1,))
                    return 0

                lax.fori_loop(ce, CK, tbody, 0)
                issue_scat(par)
        return 0

    lax.fori_loop(0, (NCK_W + 2) // 3, chunk3, 0)
    for p in range(3):
        wait_scat(p)

    @pl.when(w == 0)
    def _():
        pltpu.sync_copy(astart_v, astart_hbm)
        pltpu.sync_copy(cnt_v, cnt_hbm)


_scat = pl.kernel(
    _scat_body,
    out_type=(
        jax.ShapeDtypeStruct((EPAD,), jnp.int32),
        jax.ShapeDtypeStruct((EPAD,), jnp.int32),
        jax.ShapeDtypeStruct((NBP,), jnp.int32),
        jax.ShapeDtypeStruct((NBP,), jnp.int32),
    ),
    mesh=_mesh,
    name="sc_scat",
    scratch_types=[
        pltpu.VMEM((NW, NBP), jnp.int32),
        pltpu.VMEM((NBP,), jnp.int32),
        pltpu.VMEM((NBP,), jnp.int32),
        pltpu.VMEM((NBP,), jnp.int32),
        pltpu.VMEM((CK,), jnp.int32),
        pltpu.VMEM((CK,), jnp.int32),
        pltpu.VMEM((CK,), jnp.int32),
        pltpu.VMEM((CK,), jnp.int32),
        pltpu.VMEM((CK,), jnp.int32),
        pltpu.VMEM((CK,), jnp.int32),
        pltpu.VMEM((CK,), jnp.int32),
        pltpu.VMEM((CK,), jnp.int32),
        pltpu.VMEM((CK,), jnp.int32),
        pltpu.SemaphoreType.DMA,
        pltpu.SemaphoreType.DMA,
        pltpu.SemaphoreType.DMA,
        pltpu.SemaphoreType.DMA,
        pltpu.SemaphoreType.DMA,
        pltpu.SemaphoreType.DMA,
    ],
)


def _main_body(xl_hbm, xr_hbm, att_hbm, srcs_hbm, dsts_hbm, astart_hbm,
               cnt_hbm, o_hbm, ex_hbm, starts_v, cnt_v, attv,
               sidx0, didx0, sidx1, didx1, xlrow0, xrrow0, xlrow1, xrrow1,
               acc, den, albuf, exbuf0, exbuf1,
               semi0, semi1, semg0, semg1, semx0, semx1):
    w = _wid()
    pltpu.sync_copy(astart_hbm, starts_v)
    pltpu.sync_copy(cnt_hbm, cnt_v)
    pltpu.sync_copy(att_hbm, attv)
    attk = [attv[pl.ds(16 * k, 16)] for k in range(5)]
    zf = jnp.zeros((16,), jnp.float32)
    zi = jnp.zeros((16,), jnp.int32)
    bufs = ((sidx0, didx0, xlrow0, xrrow0, semi0, semg0, exbuf0, semx0),
            (sidx1, didx1, xlrow1, xrrow1, semi1, semg1, exbuf1, semx1))

    def clamp_idx(ref, hi):
        for kk in range(CK // 16):
            v = ref[pl.ds(16 * kk, 16)]
            ref[pl.ds(16 * kk, 16)] = jnp.minimum(jnp.maximum(v, zi), hi)

    def issue_idx(p, off):
        s = bufs[p]
        pltpu.async_copy(srcs_hbm.at[pl.ds(off, CK)], s[0], s[4])
        pltpu.async_copy(dsts_hbm.at[pl.ds(off, CK)], s[1], s[4])

    def wait_idx(p):
        s = bufs[p]
        pltpu.make_async_copy(srcs_hbm.at[pl.ds(0, CK)], s[0], s[4]).wait()
        pltpu.make_async_copy(dsts_hbm.at[pl.ds(0, CK)], s[1], s[4]).wait()

    def issue_gath2(p):
        s = bufs[p]
        clamp_idx(s[0], N - 1)
        clamp_idx(s[1], N - 1)
        pltpu.async_copy(xl_hbm.at[s[0]], s[2], s[5])
        pltpu.async_copy(xr_hbm.at[s[1]], s[3], s[5])

    def wait_gath2(p):
        s = bufs[p]
        pltpu.make_async_copy(xl_hbm.at[s[0]], s[2], s[5]).wait()
        pltpu.make_async_copy(xr_hbm.at[s[1]], s[3], s[5]).wait()

    def issue_gath1(p, off):
        s = bufs[p]
        clamp_idx(s[0], N - 1)
        pltpu.async_copy(xl_hbm.at[s[0]], s[2], s[5])
        pltpu.async_copy(ex_hbm.at[pl.ds(off, CK)], s[6], s[5])

    def wait_gath1(p):
        s = bufs[p]
        pltpu.make_async_copy(xl_hbm.at[s[0]], s[2], s[5]).wait()
        pltpu.make_async_copy(ex_hbm.at[pl.ds(0, CK)], s[6], s[5]).wait()

    def wait_ex(p):
        s = bufs[p]
        pltpu.make_async_copy(s[6], ex_hbm.at[pl.ds(0, CK)], s[7]).wait()

    def bucket_body(t, _):
        b = w + NW * t

        @pl.when(b < NB)
        def _():
            start = _sget(starts_v, b)
            n_e = _sget(cnt_v, b)
            nck = (n_e + CK - 1) >> 7
            nbase = b * BN

            def zacc(r, _):
                row = acc.at[r]
                for k in range(5):
                    row[pl.ds(16 * k, 16)] = zf
                return 0

            lax.fori_loop(0, BN, zacc, 0)
            for kk in range(BN // 16):
                den[pl.ds(16 * kk, 16)] = zf

            def coff(i):
                return pl.multiple_of(start + i * CK, CK)

            @pl.when(nck > 0)
            def _():
                issue_idx(0, coff(0))
                wait_idx(0)
                issue_gath2(0)

                @pl.when(nck > 1)
                def _():
                    issue_idx(1, coff(1))

            def p1pair(i2, _):
                for sub in range(2):
                    i = i2 * 2 + sub
                    par = sub
                    nxt = 1 - sub
                    s = bufs[par]

                    @pl.when(i < nck)
                    def _():
                        wait_gath2(par)
                        ce = jnp.minimum(CK, n_e - i * CK)
                        sdidx, sxl, sxr, sex = s[1], s[2], s[3], s[6]

                        def eb(j, _):
                            lrow = sxl.at[j]
                            rrow = sxr.at[j]
                            av = zf
                            for k in range(5):
                                sv = (lrow[pl.ds(16 * k, 16)]
                                      + rrow[pl.ds(16 * k, 16)])
                                lr = jnp.maximum(sv, NEG * sv)
                                av = av + attk[k] * lr
                            for sh in (8, 4, 2, 1):
                                av = av + av[jnp.arange(16) ^ sh]
                            albuf[pl.ds(j, 1)] = av[0:1]
                            return 0

                        lax.fori_loop(0, ce, eb, 0)

                        @pl.when(i >= 2)
                        def _():
                            wait_ex(par)
                        for kk in range(CK // 16):
                            sex[pl.ds(16 * kk, 16)] = jnp.exp(
                                albuf[pl.ds(16 * kk, 16)])
                        pltpu.async_copy(sex, ex_hbm.at[pl.ds(coff(i), CK)],
                                         s[7])

                        @pl.when(i + 1 < nck)
                        def _():
                            wait_idx(nxt)
                            issue_gath2(nxt)

                        def db(j, _):
                            dl = _sget(sdidx, j) - nbase
                            den[pl.ds(dl, 1)] = (den[pl.ds(dl, 1)]
                                                 + sex[pl.ds(j, 1)])
                            return 0

                        lax.fori_loop(0, ce, db, 0)

                        @pl.when(i + 2 < nck)
                        def _():
                            issue_idx(par, coff(i + 2))
                return 0

            lax.fori_loop(0, (nck + 1) >> 1, p1pair, 0)

            @pl.when(nck >= 1)
            def _():
                wait_ex(0)

            @pl.when(nck >= 2)
            def _():
                wait_ex(1)

            @pl.when(nck > 0)
            def _():
                issue_idx(0, coff(0))
                wait_idx(0)
                issue_gath1(0, coff(0))

                @pl.when(nck > 1)
                def _():
                    issue_idx(1, coff(1))

            def p2pair(i2, _):
                for sub in range(2):
                    i = i2 * 2 + sub
                    par = sub
                    nxt = 1 - sub
                    s = bufs[par]

                    @pl.when(i < nck)
                    def _():
                        wait_gath1(par)

                        @pl.when(i + 1 < nck)
                        def _():
                            wait_idx(nxt)
                            issue_gath1(nxt, coff(i + 1))
                        ce = jnp.minimum(CK, n_e - i * CK)
                        sdidx, sxl, sex = s[1], s[2], s[6]

                        def eb2(j, _):
                            dl = _sget(sdidx, j) - nbase
                            c1 = (sex[pl.ds(j, 1)]
                                  / (den[pl.ds(dl, 1)] + 1e-16))
                            c = c1[0]
                            arow = acc.at[dl]
                            lrow = sxl.at[j]
                            for k in range(5):
                                sl = pl.ds(16 * k, 16)
                                arow[sl] = arow[sl] + c * lrow[sl]
                            return 0

                        lax.fori_loop(0, ce, eb2, 0)

                        @pl.when(i + 2 < nck)
                        def _():
                            issue_idx(par, coff(i + 2))
                return 0

            lax.fori_loop(0, (nck + 1) >> 1, p2pair, 0)
            pltpu.sync_copy(acc, o_hbm.at[pl.ds(nbase, BN)])

        return 0

    lax.fori_loop(0, TMAX, bucket_body, 0)


_main = pl.kernel(
    _main_body,
    compiler_params=pltpu.CompilerParams(use_tc_tiling_on_sc=False),
    out_type=(
        jax.ShapeDtypeStruct((NB * BN, DP), jnp.float32),
        jax.ShapeDtypeStruct((EPAD,), jnp.float32),
    ),
    mesh=_mesh,
    name="sc_main",
    scratch_types=[
        pltpu.VMEM((NBP,), jnp.int32),
        pltpu.VMEM((NBP,), jnp.int32),
        pltpu.VMEM((DP,), jnp.float32),
        pltpu.VMEM((CK,), jnp.int32),
        pltpu.VMEM((CK,), jnp.int32),
        pltpu.VMEM((CK,), jnp.int32),
        pltpu.VMEM((CK,), jnp.int32),
        pltpu.VMEM((CK, DP), jnp.float32),
        pltpu.VMEM((CK, DP), jnp.float32),
        pltpu.VMEM((CK, DP), jnp.float32),
        pltpu.VMEM((CK, DP), jnp.float32),
        pltpu.VMEM((BN, DP), jnp.float32),
        pltpu.VMEM((BN,), jnp.float32),
        pltpu.VMEM((CK,), jnp.float32),
        pltpu.VMEM((CK,), jnp.float32),
        pltpu.VMEM((CK,), jnp.float32),
        pltpu.SemaphoreType.DMA,
        pltpu.SemaphoreType.DMA,
        pltpu.SemaphoreType.DMA,
        pltpu.SemaphoreType.DMA,
        pltpu.SemaphoreType.DMA,
        pltpu.SemaphoreType.DMA,
    ],
)


def _conv_sc(xl, xr, att80, ei):
    srcp = jnp.pad(ei[0], (0, CK))
    dstp = jnp.pad(ei[1], (0, CK))
    hist = _hist(dstp)
    srcs, dsts, astart, cnt = _scat(srcp, dstp, hist)
    o, _ = _main(xl, xr, att80, srcs, dsts, astart, cnt)
    return o


def _fin_body(of_ref, ob_ref, xf_ref, xb_ref, bf, bb, out_ref):
    f = jnp.maximum(of_ref[:, :D] + bf[:, :D] + xf_ref[...], 0.0)
    g = jnp.maximum(ob_ref[:, :D] + bb[:, :D] + xb_ref[...], 0.0)
    out_ref[...] = jnp.concatenate([f, g], axis=-1)


def _final(of, ob, xf, xb, bf, bb):
    blk = 2000
    grid = N // blk
    ospec = pl.BlockSpec((blk, DP), lambda i: (i, 0))
    xspec = pl.BlockSpec((blk, D), lambda i: (i, 0))
    bspec = pl.BlockSpec((1, DP), lambda i: (0, 0))
    return pl.pallas_call(
        _fin_body,
        grid=grid,
        in_specs=[ospec, ospec, xspec, xspec, bspec, bspec],
        out_specs=pl.BlockSpec((blk, 2 * D), lambda i: (i, 0)),
        out_shape=jax.ShapeDtypeStruct((N, 2 * D), jnp.float32),
    )(of, ob, xf, xb, bf, bb)


def kernel(x_fwd, edge_index_fwd, x_bwd, edge_index_bwd,
           Wl_f, bl_f, Wr_f, br_f, att_f, bias_f,
           Wl_b, bl_b, Wr_b, br_b, att_b, bias_b):
    padw = lambda m: jnp.pad(m, ((0, 0), (0, DP - D)))
    padv = lambda v: jnp.pad(v, (0, DP - D)).reshape(1, DP)
    xlf, xrf, xlb, xrb = _linear(
        x_fwd, x_bwd,
        padw(Wl_f), padv(bl_f), padw(Wr_f), padv(br_f),
        padw(Wl_b), padv(bl_b), padw(Wr_b), padv(br_b))
    of = _conv_sc(xlf, xrf, jnp.pad(att_f, (0, DP - D)), edge_index_fwd)
    ob = _conv_sc(xlb, xrb, jnp.pad(att_b, (0, DP - D)), edge_index_bwd)
    return _final(of, ob, x_fwd, x_bwd, padv(bias_f), padv(bias_b))


# speedup vs baseline: 4.2792x; 1.2522x over previous
import jax
import jax.numpy as jnp
from jax import lax
from jax.experimental import pallas as pl
from jax.experimental.pallas import tpu as pltpu
from jax.experimental.pallas import tpu_sc as plsc

N = 50000
D = 73
DP = 80
E = 800000
NEG = 0.2
BSH = 8
BN = 256
NB = (N + BN - 1) // BN
NBP = 256
NW = 32
EW = E // NW
CK = 128
NCK_W = (EW + CK - 1) // CK
ESRT = E + CK * NB
EPAD = ESRT + CK
TMAX = (NB + NW - 1) // NW

_mesh = plsc.VectorSubcoreMesh(core_axis_name="c", subcore_axis_name="s")


def _wid():
    return lax.axis_index("s") * 2 + lax.axis_index("c")


def _sget(ref, i):
    return ref[pl.ds(i, 1)][0]


def _lin_body(xf_ref, xb_ref, wlf, blf, wrf, brf, wlb, blb, wrb, brb,
              xlf_ref, xrf_ref, xlb_ref, xrb_ref):
    xf = xf_ref[...]
    xb = xb_ref[...]
    xlf_ref[...] = jnp.dot(xf, wlf[...], preferred_element_type=jnp.float32) + blf[...]
    xrf_ref[...] = jnp.dot(xf, wrf[...], preferred_element_type=jnp.float32) + brf[...]
    xlb_ref[...] = jnp.dot(xb, wlb[...], preferred_element_type=jnp.float32) + blb[...]
    xrb_ref[...] = jnp.dot(xb, wrb[...], preferred_element_type=jnp.float32) + brb[...]


def _linear(xf, xb, wlf, blf, wrf, brf, wlb, blb, wrb, brb):
    blk = 2000
    grid = N // blk
    xspec = pl.BlockSpec((blk, D), lambda i: (i, 0))
    wspec = pl.BlockSpec((D, DP), lambda i: (0, 0))
    bspec = pl.BlockSpec((1, DP), lambda i: (0, 0))
    ospec = pl.BlockSpec((blk, DP), lambda i: (i, 0))
    return pl.pallas_call(
        _lin_body,
        grid=grid,
        in_specs=[xspec, xspec, wspec, bspec, wspec, bspec,
                  wspec, bspec, wspec, bspec],
        out_specs=[ospec, ospec, ospec, ospec],
        out_shape=[jax.ShapeDtypeStruct((N, DP), jnp.float32)] * 4,
    )(xf, xb, wlf, blf, wrf, brf, wlb, blb, wrb, brb)


def _hist_body(dst_hbm, hist_hbm, dbuf, hv, hs):
    w = _wid()
    pltpu.sync_copy(dst_hbm.at[pl.ds(pl.multiple_of(w * EW, 8), EW)], dbuf)

    def zb(b, _):
        hs[b] = 0
        return 0

    lax.fori_loop(0, NBP, zb, 0)

    def egroup(g, _):
        dv = dbuf[pl.ds(g * 16, 16)] >> BSH
        for jj in range(16):
            b = dv[jj]
            hs[b] = hs[b] + 1
        return 0

    lax.fori_loop(0, EW >> 4, egroup, 0)

    def body(e, _):
        b = _sget(dbuf, e) >> BSH
        hs[b] = hs[b] + 1
        return 0

    lax.fori_loop((EW >> 4) << 4, EW, body, 0)

    def cp(b, _):
        hv[pl.ds(b, 1)] = jnp.reshape(hs[b], (1,))
        return 0

    lax.fori_loop(0, NBP, cp, 0)
    pltpu.sync_copy(hv, hist_hbm.at[w])


_hist = pl.kernel(
    _hist_body,
    out_type=jax.ShapeDtypeStruct((NW, NBP), jnp.int32),
    mesh=_mesh,
    name="sc_hist",
    scratch_types=[
        pltpu.VMEM((EW,), jnp.int32),
        pltpu.VMEM((NBP,), jnp.int32),
        pltpu.SMEM((NBP,), jnp.int32),
    ],
)


def _scat_body(src_hbm, dst_hbm, hist_hbm, srcs_hbm, dsts_hbm,
               astart_hbm, cnt_hbm, hall, cur, astart_v, cnt_v,
               sbuf, dbuf, posb, sbuf1, dbuf1, posb1, sbuf2, dbuf2, posb2,
               curs, semr, semw, semr1, semw1, semr2, semw2):
    w = _wid()
    pltpu.sync_copy(hist_hbm, hall)
    zi = jnp.zeros((16,), jnp.int32)

    for kk in range(NBP // 16):
        sl = pl.ds(16 * kk, 16)

        def ws(i, c):
            return c + hall[i, sl]

        cnt_v[sl] = lax.fori_loop(0, NW, ws, zi)
        cur[sl] = lax.fori_loop(0, w, ws, zi)

    def bloop(b, a):
        cs = _sget(cnt_v, b)
        astart_v[pl.ds(b, 1)] = jnp.reshape(a, (1,))
        curs[b] = _sget(cur, b) + a
        return (a + cs + CK - 1) & (-CK)

    total = lax.fori_loop(0, NB, bloop, 0)
    astart_v[pl.ds(NB, 1)] = jnp.reshape(total, (1,))

    base = w * EW
    bufs = ((sbuf, dbuf, posb, semr, semw),
            (sbuf1, dbuf1, posb1, semr1, semw1),
            (sbuf2, dbuf2, posb2, semr2, semw2))

    def issue_read(p, i):
        off = pl.multiple_of(base + i * CK, 8)
        s = bufs[p]
        pltpu.async_copy(src_hbm.at[pl.ds(off, CK)], s[0], s[3])
        pltpu.async_copy(dst_hbm.at[pl.ds(off, CK)], s[1], s[3])

    def wait_read(p):
        s = bufs[p]
        pltpu.make_async_copy(src_hbm.at[pl.ds(0, CK)], s[0], s[3]).wait()
        pltpu.make_async_copy(dst_hbm.at[pl.ds(0, CK)], s[1], s[3]).wait()

    def issue_scat(p):
        s = bufs[p]
        pltpu.async_copy(s[0], srcs_hbm.at[s[2]], s[4])
        pltpu.async_copy(s[1], dsts_hbm.at[s[2]], s[4])

    def wait_scat(p):
        s = bufs[p]
        pltpu.make_async_copy(s[0], srcs_hbm.at[s[2]], s[4]).wait()
        pltpu.make_async_copy(s[1], dsts_hbm.at[s[2]], s[4]).wait()

    issue_read(0, 0)

    def chunk3(i3, _):
        for sub in range(3):
            i = i3 * 3 + sub
            par = sub
            nxt = (sub + 1) % 3

            @pl.when(i < NCK_W)
            def _():
                @pl.when(i + 1 < NCK_W)
                def _():
                    @pl.when(i >= 2)
                    def _():
                        wait_scat(nxt)
                    issue_read(nxt, i + 1)
                wait_read(par)
                s = bufs[par]
                sdbuf, sposb = s[1], s[2]
                ce = jnp.minimum(CK, EW - i * CK)

                def egroup(g, _):
                    dv = sdbuf[pl.ds(g * 16, 16)] >> BSH
                    for jj in range(16):
                        b = dv[jj]
                        p = curs[b]
                        curs[b] = p + 1
                        sposb[pl.ds(g * 16 + jj, 1)] = jnp.reshape(p, (1,))
                    return 0

                lax.fori_loop(0, ce >> 4, egroup, 0)

                def ebody(j, _):
                    b = _sget(sdbuf, j) >> BSH
                    p = curs[b]
                    curs[b] = p + 1
                    sposb[pl.ds(j, 1)] = jnp.reshape(p, (1,))
                    return 0

                lax.fori_loop((ce >> 4) << 4, ce, ebody, 0)

                def tbody(j, _):
                    sposb[pl.ds(j, 1)] = jnp.reshape(ESRT + j, (1,))
                    return 0

                lax.fori_loop(ce, CK, tbody, 0)
                issue_scat(par)
        return 0

    lax.fori_loop(0, (NCK_W + 2) // 3, chunk3, 0)
    for p in range(3):
        wait_scat(p)

    @pl.when(w == 0)
    def _():
        pltpu.sync_copy(astart_v, astart_hbm)
        pltpu.sync_copy(cnt_v, cnt_hbm)


_scat = pl.kernel(
    _scat_body,
    out_type=(
        jax.ShapeDtypeStruct((EPAD,), jnp.int32),
        jax.ShapeDtypeStruct((EPAD,), jnp.int32),
        jax.ShapeDtypeStruct((NBP,), jnp.int32),
        jax.ShapeDtypeStruct((NBP,), jnp.int32),
    ),
    mesh=_mesh,
    name="sc_scat",
    scratch_types=[
        pltpu.VMEM((NW, NBP), jnp.int32),
        pltpu.VMEM((NBP,), jnp.int32),
        pltpu.VMEM((NBP,), jnp.int32),
        pltpu.VMEM((NBP,), jnp.int32),
        pltpu.VMEM((CK,), jnp.int32),
        pltpu.VMEM((CK,), jnp.int32),
        pltpu.VMEM((CK,), jnp.int32),
        pltpu.VMEM((CK,), jnp.int32),
        pltpu.VMEM((CK,), jnp.int32),
        pltpu.VMEM((CK,), jnp.int32),
        pltpu.VMEM((CK,), jnp.int32),
        pltpu.VMEM((CK,), jnp.int32),
        pltpu.VMEM((CK,), jnp.int32),
        pltpu.SMEM((NBP,), jnp.int32),
        pltpu.SemaphoreType.DMA,
        pltpu.SemaphoreType.DMA,
        pltpu.SemaphoreType.DMA,
        pltpu.SemaphoreType.DMA,
        pltpu.SemaphoreType.DMA,
        pltpu.SemaphoreType.DMA,
    ],
)


def _main_body(xl_hbm, xr_hbm, att_hbm, srcs_hbm, dsts_hbm, astart_hbm,
               cnt_hbm, o_hbm, ex_hbm, starts_v, cnt_v, attv,
               sidx0, didx0, sidx1, didx1, xlrow0, xrrow0, xlrow1, xrrow1,
               acc, den, den1, den2, den3, albuf, exbuf0, exbuf1,
               semi0, semi1, semg0, semg1, semx0, semx1):
    w = _wid()
    pltpu.sync_copy(astart_hbm, starts_v)
    pltpu.sync_copy(cnt_hbm, cnt_v)
    pltpu.sync_copy(att_hbm, attv)
    attk = [attv[pl.ds(16 * k, 16)] for k in range(5)]
    zf = jnp.zeros((16,), jnp.float32)
    zi = jnp.zeros((16,), jnp.int32)
    bufs = ((sidx0, didx0, xlrow0, xrrow0, semi0, semg0, exbuf0, semx0),
            (sidx1, didx1, xlrow1, xrrow1, semi1, semg1, exbuf1, semx1))

    def clamp_idx(ref, hi):
        for kk in range(CK // 16):
            v = ref[pl.ds(16 * kk, 16)]
            ref[pl.ds(16 * kk, 16)] = jnp.minimum(jnp.maximum(v, zi), hi)

    def issue_idx(p, off):
        s = bufs[p]
        pltpu.async_copy(srcs_hbm.at[pl.ds(off, CK)], s[0], s[4])
        pltpu.async_copy(dsts_hbm.at[pl.ds(off, CK)], s[1], s[4])

    def wait_idx(p):
        s = bufs[p]
        pltpu.make_async_copy(srcs_hbm.at[pl.ds(0, CK)], s[0], s[4]).wait()
        pltpu.make_async_copy(dsts_hbm.at[pl.ds(0, CK)], s[1], s[4]).wait()

    def issue_gath2(p):
        s = bufs[p]
        clamp_idx(s[0], N - 1)
        clamp_idx(s[1], N - 1)
        pltpu.async_copy(xl_hbm.at[s[0]], s[2], s[5])
        pltpu.async_copy(xr_hbm.at[s[1]], s[3], s[5])

    def wait_gath2(p):
        s = bufs[p]
        pltpu.make_async_copy(xl_hbm.at[s[0]], s[2], s[5]).wait()
        pltpu.make_async_copy(xr_hbm.at[s[1]], s[3], s[5]).wait()

    def issue_gath1(p, off):
        s = bufs[p]
        clamp_idx(s[0], N - 1)
        pltpu.async_copy(xl_hbm.at[s[0]], s[2], s[5])
        pltpu.async_copy(ex_hbm.at[pl.ds(off, CK)], s[6], s[5])

    def wait_gath1(p):
        s = bufs[p]
        pltpu.make_async_copy(xl_hbm.at[s[0]], s[2], s[5]).wait()
        pltpu.make_async_copy(ex_hbm.at[pl.ds(0, CK)], s[6], s[5]).wait()

    def wait_ex(p):
        s = bufs[p]
        pltpu.make_async_copy(s[6], ex_hbm.at[pl.ds(0, CK)], s[7]).wait()

    def bucket_body(t, _):
        b = w + NW * t

        @pl.when(b < NB)
        def _():
            start = _sget(starts_v, b)
            n_e = _sget(cnt_v, b)
            nck = (n_e + CK - 1) >> 7
            nbase = b * BN

            def zacc(r, _):
                row = acc.at[r]
                for k in range(5):
                    row[pl.ds(16 * k, 16)] = zf
                return 0

            lax.fori_loop(0, BN, zacc, 0)
            for kk in range(BN // 16):
                den[pl.ds(16 * kk, 16)] = zf
                den1[pl.ds(16 * kk, 16)] = zf
                den2[pl.ds(16 * kk, 16)] = zf
                den3[pl.ds(16 * kk, 16)] = zf

            def coff(i):
                return pl.multiple_of(start + i * CK, CK)

            @pl.when(nck > 0)
            def _():
                issue_idx(0, coff(0))
                wait_idx(0)
                issue_gath2(0)

                @pl.when(nck > 1)
                def _():
                    issue_idx(1, coff(1))

            def p1pair(i2, _):
                for sub in range(2):
                    i = i2 * 2 + sub
                    par = sub
                    nxt = 1 - sub
                    s = bufs[par]

                    @pl.when(i < nck)
                    def _():
                        wait_gath2(par)
                        ce = jnp.minimum(CK, n_e - i * CK)
                        sdidx, sxl, sxr, sex = s[1], s[2], s[3], s[6]

                        def eb(j, _):
                            lrow = sxl.at[j]
                            rrow = sxr.at[j]
                            av = zf
                            for k in range(5):
                                sv = (lrow[pl.ds(16 * k, 16)]
                                      + rrow[pl.ds(16 * k, 16)])
                                lr = jnp.maximum(sv, NEG * sv)
                                av = av + attk[k] * lr
                            for sh in (8, 4, 2, 1):
                                av = av + av[jnp.arange(16) ^ sh]
                            albuf[pl.ds(j, 1)] = av[0:1]
                            return 0

                        lax.fori_loop(0, ce, eb, 0)

                        @pl.when(i >= 2)
                        def _():
                            wait_ex(par)
                        for kk in range(CK // 16):
                            sex[pl.ds(16 * kk, 16)] = jnp.exp(
                                albuf[pl.ds(16 * kk, 16)])
                        pltpu.async_copy(sex, ex_hbm.at[pl.ds(coff(i), CK)],
                                         s[7])

                        @pl.when(i + 1 < nck)
                        def _():
                            wait_idx(nxt)
                            issue_gath2(nxt)

                        dens = (den, den1, den2, den3)

                        def dbg(g, _):
                            dv = sdidx[pl.ds(g * 16, 16)] - nbase
                            for jj in range(16):
                                dl = dv[jj]
                                dq = dens[jj % 4]
                                dq[pl.ds(dl, 1)] = (dq[pl.ds(dl, 1)]
                                                    + sex[pl.ds(g * 16 + jj,
                                                                1)])
                            return 0

                        lax.fori_loop(0, ce >> 4, dbg, 0)

                        def db(j, _):
                            dl = _sget(sdidx, j) - nbase
                            den[pl.ds(dl, 1)] = (den[pl.ds(dl, 1)]
                                                 + sex[pl.ds(j, 1)])
                            return 0

                        lax.fori_loop((ce >> 4) << 4, ce, db, 0)

                        @pl.when(i + 2 < nck)
                        def _():
                            issue_idx(par, coff(i + 2))
                return 0

            lax.fori_loop(0, (nck + 1) >> 1, p1pair, 0)

            @pl.when(nck >= 1)
            def _():
                wait_ex(0)

            @pl.when(nck >= 2)
            def _():
                wait_ex(1)

            one = jnp.full((16,), 1.0, jnp.float32)
            for kk in range(BN // 16):
                sl = pl.ds(16 * kk, 16)
                den[sl] = one / (den[sl] + den1[sl] + den2[sl] + den3[sl]
                                 + 1e-16)

            @pl.when(nck > 0)
            def _():
                issue_idx(0, coff(0))
                wait_idx(0)
                issue_gath1(0, coff(0))

                @pl.when(nck > 1)
                def _():
                    issue_idx(1, coff(1))

            def p2pair(i2, _):
                for sub in range(2):
                    i = i2 * 2 + sub
                    par = sub
                    nxt = 1 - sub
                    s = bufs[par]

                    @pl.when(i < nck)
                    def _():
                        wait_gath1(par)

                        @pl.when(i + 1 < nck)
                        def _():
                            wait_idx(nxt)
                            issue_gath1(nxt, coff(i + 1))
                        ce = jnp.minimum(CK, n_e - i * CK)
                        sdidx, sxl, sex = s[1], s[2], s[6]

                        def eb2g(g, _):
                            dv = sdidx[pl.ds(g * 16, 16)] - nbase
                            for jj in range(16):
                                j = g * 16 + jj
                                dl = dv[jj]
                                c1 = sex[pl.ds(j, 1)] * den[pl.ds(dl, 1)]
                                c = c1[0]
                                arow = acc.at[dl]
                                lrow = sxl.at[j]
                                for k in range(5):
                                    sl = pl.ds(16 * k, 16)
                                    arow[sl] = arow[sl] + c * lrow[sl]
                            return 0

                        lax.fori_loop(0, ce >> 4, eb2g, 0)

                        def eb2(j, _):
                            dl = _sget(sdidx, j) - nbase
                            c1 = sex[pl.ds(j, 1)] * den[pl.ds(dl, 1)]
                            c = c1[0]
                            arow = acc.at[dl]
                            lrow = sxl.at[j]
                            for k in range(5):
                                sl = pl.ds(16 * k, 16)
                                arow[sl] = arow[sl] + c * lrow[sl]
                            return 0

                        lax.fori_loop((ce >> 4) << 4, ce, eb2, 0)

                        @pl.when(i + 2 < nck)
                        def _():
                            issue_idx(par, coff(i + 2))
                return 0

            lax.fori_loop(0, (nck + 1) >> 1, p2pair, 0)
            pltpu.sync_copy(acc, o_hbm.at[pl.ds(nbase, BN)])

        return 0

    lax.fori_loop(0, TMAX, bucket_body, 0)


_main = pl.kernel(
    _main_body,
    compiler_params=pltpu.CompilerParams(use_tc_tiling_on_sc=False),
    out_type=(
        jax.ShapeDtypeStruct((NB * BN, DP), jnp.float32),
        jax.ShapeDtypeStruct((EPAD,), jnp.float32),
    ),
    mesh=_mesh,
    name="sc_main",
    scratch_types=[
        pltpu.VMEM((NBP,), jnp.int32),
        pltpu.VMEM((NBP,), jnp.int32),
        pltpu.VMEM((DP,), jnp.float32),
        pltpu.VMEM((CK,), jnp.int32),
        pltpu.VMEM((CK,), jnp.int32),
        pltpu.VMEM((CK,), jnp.int32),
        pltpu.VMEM((CK,), jnp.int32),
        pltpu.VMEM((CK, DP), jnp.float32),
        pltpu.VMEM((CK, DP), jnp.float32),
        pltpu.VMEM((CK, DP), jnp.float32),
        pltpu.VMEM((CK, DP), jnp.float32),
        pltpu.VMEM((BN, DP), jnp.float32),
        pltpu.VMEM((BN,), jnp.float32),
        pltpu.VMEM((BN,), jnp.float32),
        pltpu.VMEM((BN,), jnp.float32),
        pltpu.VMEM((BN,), jnp.float32),
        pltpu.VMEM((CK,), jnp.float32),
        pltpu.VMEM((CK,), jnp.float32),
        pltpu.VMEM((CK,), jnp.float32),
        pltpu.SemaphoreType.DMA,
        pltpu.SemaphoreType.DMA,
        pltpu.SemaphoreType.DMA,
        pltpu.SemaphoreType.DMA,
        pltpu.SemaphoreType.DMA,
        pltpu.SemaphoreType.DMA,
    ],
)


def _conv_sc(xl, xr, att80, ei):
    srcp = jnp.pad(ei[0], (0, CK))
    dstp = jnp.pad(ei[1], (0, CK))
    hist = _hist(dstp)
    srcs, dsts, astart, cnt = _scat(srcp, dstp, hist)
    o, _ = _main(xl, xr, att80, srcs, dsts, astart, cnt)
    return o


def _fin_body(of_ref, ob_ref, xf_ref, xb_ref, bf, bb, out_ref):
    f = jnp.maximum(of_ref[:, :D] + bf[:, :D] + xf_ref[...], 0.0)
    g = jnp.maximum(ob_ref[:, :D] + bb[:, :D] + xb_ref[...], 0.0)
    out_ref[...] = jnp.concatenate([f, g], axis=-1)


def _final(of, ob, xf, xb, bf, bb):
    blk = 2000
    grid = N // blk
    ospec = pl.BlockSpec((blk, DP), lambda i: (i, 0))
    xspec = pl.BlockSpec((blk, D), lambda i: (i, 0))
    bspec = pl.BlockSpec((1, DP), lambda i: (0, 0))
    return pl.pallas_call(
        _fin_body,
        grid=grid,
        in_specs=[ospec, ospec, xspec, xspec, bspec, bspec],
        out_specs=pl.BlockSpec((blk, 2 * D), lambda i: (i, 0)),
        out_shape=jax.ShapeDtypeStruct((N, 2 * D), jnp.float32),
    )(of, ob, xf, xb, bf, bb)


def kernel(x_fwd, edge_index_fwd, x_bwd, edge_index_bwd,
           Wl_f, bl_f, Wr_f, br_f, att_f, bias_f,
           Wl_b, bl_b, Wr_b, br_b, att_b, bias_b):
    padw = lambda m: jnp.pad(m, ((0, 0), (0, DP - D)))
    padv = lambda v: jnp.pad(v, (0, DP - D)).reshape(1, DP)
    xlf, xrf, xlb, xrb = _linear(
        x_fwd, x_bwd,
        padw(Wl_f), padv(bl_f), padw(Wr_f), padv(br_f),
        padw(Wl_b), padv(bl_b), padw(Wr_b), padv(br_b))
    of = _conv_sc(xlf, xrf, jnp.pad(att_f, (0, DP - D)), edge_index_fwd)
    ob = _conv_sc(xlb, xrb, jnp.pad(att_b, (0, DP - D)), edge_index_bwd)
    return _final(of, ob, x_fwd, x_bwd, padv(bias_f), padv(bias_b))


# speedup vs baseline: 7.3791x; 1.7244x over previous
import jax
import jax.numpy as jnp
from jax import lax
from jax.experimental import pallas as pl
from jax.experimental.pallas import tpu as pltpu
from jax.experimental.pallas import tpu_sc as plsc

N = 50000
D = 73
DP = 80
E = 800000
NEG = 0.2
BSH = 8
BN = 256
NB = (N + BN - 1) // BN
NBP = 256
NW = 32
EW = E // NW
CK = 128
NCK_W = (EW + CK - 1) // CK
SPCAP = E // 2 + CK * NB + CK
EPAD = 2 * SPCAP + CK
TMAX = (NB + NW - 1) // NW

_mesh = plsc.VectorSubcoreMesh(core_axis_name="c", subcore_axis_name="s")


def _wid():
    return lax.axis_index("s") * 2 + lax.axis_index("c")


def _sget(ref, i):
    return ref[pl.ds(i, 1)][0]


def _lin_body(xf_ref, xb_ref, wlf, blf, wrf, brf, wlb, blb, wrb, brb,
              xlf_ref, xrf_ref, xlb_ref, xrb_ref):
    xf = xf_ref[...]
    xb = xb_ref[...]
    xlf_ref[...] = jnp.dot(xf, wlf[...], preferred_element_type=jnp.float32) + blf[...]
    xrf_ref[...] = jnp.dot(xf, wrf[...], preferred_element_type=jnp.float32) + brf[...]
    xlb_ref[...] = jnp.dot(xb, wlb[...], preferred_element_type=jnp.float32) + blb[...]
    xrb_ref[...] = jnp.dot(xb, wrb[...], preferred_element_type=jnp.float32) + brb[...]


def _linear(xf, xb, wlf, blf, wrf, brf, wlb, blb, wrb, brb):
    blk = 2000
    grid = N // blk
    xspec = pl.BlockSpec((blk, D), lambda i: (i, 0))
    wspec = pl.BlockSpec((D, DP), lambda i: (0, 0))
    bspec = pl.BlockSpec((1, DP), lambda i: (0, 0))
    ospec = pl.BlockSpec((blk, DP), lambda i: (i, 0))
    return pl.pallas_call(
        _lin_body,
        grid=grid,
        in_specs=[xspec, xspec, wspec, bspec, wspec, bspec,
                  wspec, bspec, wspec, bspec],
        out_specs=[ospec, ospec, ospec, ospec],
        out_shape=[jax.ShapeDtypeStruct((N, DP), jnp.float32)] * 4,
    )(xf, xb, wlf, blf, wrf, brf, wlb, blb, wrb, brb)


def _hist_body(dst_hbm, hist_hbm, dbuf, hv, hs):
    w = _wid()
    pltpu.sync_copy(dst_hbm.at[pl.ds(pl.multiple_of(w * EW, 8), EW)], dbuf)

    def zb(b, _):
        hs[b] = 0
        return 0

    lax.fori_loop(0, NBP, zb, 0)

    def egroup(g, _):
        dv = dbuf[pl.ds(g * 16, 16)] >> BSH
        for jj in range(16):
            b = dv[jj]
            hs[b] = hs[b] + 1
        return 0

    lax.fori_loop(0, EW >> 4, egroup, 0)

    def body(e, _):
        b = _sget(dbuf, e) >> BSH
        hs[b] = hs[b] + 1
        return 0

    lax.fori_loop((EW >> 4) << 4, EW, body, 0)

    def cp(b, _):
        hv[pl.ds(b, 1)] = jnp.reshape(hs[b], (1,))
        return 0

    lax.fori_loop(0, NBP, cp, 0)
    pltpu.sync_copy(hv, hist_hbm.at[w])


_hist = pl.kernel(
    _hist_body,
    out_type=jax.ShapeDtypeStruct((NW, NBP), jnp.int32),
    mesh=_mesh,
    name="sc_hist",
    scratch_types=[
        pltpu.VMEM((EW,), jnp.int32),
        pltpu.VMEM((NBP,), jnp.int32),
        pltpu.SMEM((NBP,), jnp.int32),
    ],
)


def _scat_body(src_hbm, dst_hbm, hist_hbm, srcs_hbm, dsts_hbm,
               a0_hbm, c0_hbm, a1_hbm, c1_hbm,
               hall, cur, a0_v, c0_v, a1_v, c1_v,
               sbuf, dbuf, posb, sbuf1, dbuf1, posb1, sbuf2, dbuf2, posb2,
               srcs_sp, dsts_sp, curs,
               semr, semw, semr1, semw1, semr2, semw2):
    w = _wid()
    h = w & 1
    sid = lax.axis_index("s")
    pltpu.sync_copy(hist_hbm, hall)
    zi = jnp.zeros((16,), jnp.int32)

    for kk in range(NBP // 16):
        sl = pl.ds(16 * kk, 16)

        def ws0(k, c):
            return c + hall[2 * k, sl]

        def ws1(k, c):
            return c + hall[2 * k + 1, sl]

        def wsm(k, c):
            return c + hall[2 * k + h, sl]

        c0_v[sl] = lax.fori_loop(0, 16, ws0, zi)
        c1_v[sl] = lax.fori_loop(0, 16, ws1, zi)
        cur[sl] = lax.fori_loop(0, sid, wsm, zi)

    def bloop0(b, a):
        cs = _sget(c0_v, b)
        a0_v[pl.ds(b, 1)] = jnp.reshape(a, (1,))
        return (a + cs + CK - 1) & (-CK)

    lax.fori_loop(0, NB, bloop0, 0)

    def bloop1(b, a):
        cs = _sget(c1_v, b)
        a1_v[pl.ds(b, 1)] = jnp.reshape(a, (1,))
        return (a + cs + CK - 1) & (-CK)

    lax.fori_loop(0, NB, bloop1, 0)

    def cinit(b, _):
        ab = jnp.where(h == 0, _sget(a0_v, b), _sget(a1_v, b))
        curs[b] = ab + _sget(cur, b)
        return 0

    lax.fori_loop(0, NB, cinit, 0)

    base = w * EW
    bufs = ((sbuf, dbuf, posb, semr, semw),
            (sbuf1, dbuf1, posb1, semr1, semw1),
            (sbuf2, dbuf2, posb2, semr2, semw2))

    def issue_read(p, i):
        off = pl.multiple_of(base + i * CK, 8)
        s = bufs[p]
        pltpu.async_copy(src_hbm.at[pl.ds(off, CK)], s[0], s[3])
        pltpu.async_copy(dst_hbm.at[pl.ds(off, CK)], s[1], s[3])

    def wait_read(p):
        s = bufs[p]
        pltpu.make_async_copy(src_hbm.at[pl.ds(0, CK)], s[0], s[3]).wait()
        pltpu.make_async_copy(dst_hbm.at[pl.ds(0, CK)], s[1], s[3]).wait()

    def issue_scat(p):
        s = bufs[p]
        pltpu.async_copy(s[0], srcs_sp.at[s[2]], s[4])
        pltpu.async_copy(s[1], dsts_sp.at[s[2]], s[4])

    def wait_scat(p):
        s = bufs[p]
        pltpu.make_async_copy(s[0], srcs_sp.at[s[2]], s[4]).wait()
        pltpu.make_async_copy(s[1], dsts_sp.at[s[2]], s[4]).wait()

    issue_read(0, 0)

    def chunk3(i3, _):
        for sub in range(3):
            i = i3 * 3 + sub
            par = sub
            nxt = (sub + 1) % 3

            @pl.when(i < NCK_W)
            def _():
                @pl.when(i + 1 < NCK_W)
                def _():
                    @pl.when(i >= 2)
                    def _():
                        wait_scat(nxt)
                    issue_read(nxt, i + 1)
                wait_read(par)
                s = bufs[par]
                sdbuf, sposb = s[1], s[2]
                ce = jnp.minimum(CK, EW - i * CK)

                def egroup(g, _):
                    dv = sdbuf[pl.ds(g * 16, 16)] >> BSH
                    for jj in range(16):
                        b = dv[jj]
                        p = curs[b]
                        curs[b] = p + 1
                        sposb[pl.ds(g * 16 + jj, 1)] = jnp.reshape(p, (1,))
                    return 0

                lax.fori_loop(0, ce >> 4, egroup, 0)

                def ebody(j, _):
                    b = _sget(sdbuf, j) >> BSH
                    p = curs[b]
                    curs[b] = p + 1
                    sposb[pl.ds(j, 1)] = jnp.reshape(p, (1,))
                    return 0

                lax.fori_loop((ce >> 4) << 4, ce, ebody, 0)

                def tbody(j, _):
                    sposb[pl.ds(j, 1)] = jnp.reshape(SPCAP + j, (1,))
                    return 0

                lax.fori_loop(ce, CK, tbody, 0)
                issue_scat(par)
        return 0

    lax.fori_loop(0, (NCK_W + 2) // 3, chunk3, 0)
    for p in range(3):
        wait_scat(p)
    plsc.subcore_barrier()

    @pl.when(sid == 0)
    def _():
        off = pl.multiple_of(h * SPCAP, 8)
        pltpu.async_copy(srcs_sp.at[pl.ds(0, SPCAP)],
                         srcs_hbm.at[pl.ds(off, SPCAP)], semr)
        pltpu.async_copy(dsts_sp.at[pl.ds(0, SPCAP)],
                         dsts_hbm.at[pl.ds(off, SPCAP)], semr)
        pltpu.make_async_copy(srcs_sp.at[pl.ds(0, SPCAP)],
                              srcs_hbm.at[pl.ds(off, SPCAP)], semr).wait()
        pltpu.make_async_copy(dsts_sp.at[pl.ds(0, SPCAP)],
                              dsts_hbm.at[pl.ds(off, SPCAP)], semr).wait()

    @pl.when(w == 0)
    def _():
        pltpu.sync_copy(a0_v, a0_hbm)
        pltpu.sync_copy(c0_v, c0_hbm)
        pltpu.sync_copy(a1_v, a1_hbm)
        pltpu.sync_copy(c1_v, c1_hbm)


_scat = pl.kernel(
    _scat_body,
    out_type=(
        jax.ShapeDtypeStruct((EPAD,), jnp.int32),
        jax.ShapeDtypeStruct((EPAD,), jnp.int32),
        jax.ShapeDtypeStruct((NBP,), jnp.int32),
        jax.ShapeDtypeStruct((NBP,), jnp.int32),
        jax.ShapeDtypeStruct((NBP,), jnp.int32),
        jax.ShapeDtypeStruct((NBP,), jnp.int32),
    ),
    mesh=_mesh,
    name="sc_scat",
    scratch_types=[
        pltpu.VMEM((NW, NBP), jnp.int32),
        pltpu.VMEM((NBP,), jnp.int32),
        pltpu.VMEM((NBP,), jnp.int32),
        pltpu.VMEM((NBP,), jnp.int32),
        pltpu.VMEM((NBP,), jnp.int32),
        pltpu.VMEM((NBP,), jnp.int32),
        pltpu.VMEM((CK,), jnp.int32),
        pltpu.VMEM((CK,), jnp.int32),
        pltpu.VMEM((CK,), jnp.int32),
        pltpu.VMEM((CK,), jnp.int32),
        pltpu.VMEM((CK,), jnp.int32),
        pltpu.VMEM((CK,), jnp.int32),
        pltpu.VMEM((CK,), jnp.int32),
        pltpu.VMEM((CK,), jnp.int32),
        pltpu.VMEM((CK,), jnp.int32),
        pltpu.VMEM_SHARED((SPCAP + CK,), jnp.int32),
        pltpu.VMEM_SHARED((SPCAP + CK,), jnp.int32),
        pltpu.SMEM((NBP,), jnp.int32),
        pltpu.SemaphoreType.DMA,
        pltpu.SemaphoreType.DMA,
        pltpu.SemaphoreType.DMA,
        pltpu.SemaphoreType.DMA,
        pltpu.SemaphoreType.DMA,
        pltpu.SemaphoreType.DMA,
    ],
)


def _main_body(xl_hbm, xr_hbm, att_hbm, srcs_hbm, dsts_hbm, a0_hbm,
               c0_hbm, a1_hbm, c1_hbm, o_hbm, ex_hbm,
               s0_v, n0_v, s1_v, n1_v, attv,
               sidx0, didx0, sidx1, didx1, xlrow0, xrrow0, xlrow1, xrrow1,
               acc, den, den1, den2, den3, albuf, exbuf0, exbuf1,
               semi0, semi1, semg0, semg1, semx0, semx1):
    w = _wid()
    pltpu.sync_copy(a0_hbm, s0_v)
    pltpu.sync_copy(c0_hbm, n0_v)
    pltpu.sync_copy(a1_hbm, s1_v)
    pltpu.sync_copy(c1_hbm, n1_v)
    pltpu.sync_copy(att_hbm, attv)
    attk = [attv[pl.ds(16 * k, 16)] for k in range(5)]
    zf = jnp.zeros((16,), jnp.float32)
    zi = jnp.zeros((16,), jnp.int32)
    bufs = ((sidx0, didx0, xlrow0, xrrow0, semi0, semg0, exbuf0, semx0),
            (sidx1, didx1, xlrow1, xrrow1, semi1, semg1, exbuf1, semx1))

    def clamp_idx(ref, hi):
        for kk in range(CK // 16):
            v = ref[pl.ds(16 * kk, 16)]
            ref[pl.ds(16 * kk, 16)] = jnp.minimum(jnp.maximum(v, zi), hi)

    def issue_idx(p, off):
        s = bufs[p]
        pltpu.async_copy(srcs_hbm.at[pl.ds(off, CK)], s[0], s[4])
        pltpu.async_copy(dsts_hbm.at[pl.ds(off, CK)], s[1], s[4])

    def wait_idx(p):
        s = bufs[p]
        pltpu.make_async_copy(srcs_hbm.at[pl.ds(0, CK)], s[0], s[4]).wait()
        pltpu.make_async_copy(dsts_hbm.at[pl.ds(0, CK)], s[1], s[4]).wait()

    def issue_gath2(p):
        s = bufs[p]
        clamp_idx(s[0], N - 1)
        clamp_idx(s[1], N - 1)
        pltpu.async_copy(xl_hbm.at[s[0]], s[2], s[5])
        pltpu.async_copy(xr_hbm.at[s[1]], s[3], s[5])

    def wait_gath2(p):
        s = bufs[p]
        pltpu.make_async_copy(xl_hbm.at[s[0]], s[2], s[5]).wait()
        pltpu.make_async_copy(xr_hbm.at[s[1]], s[3], s[5]).wait()

    def issue_gath1(p, off):
        s = bufs[p]
        clamp_idx(s[0], N - 1)
        pltpu.async_copy(xl_hbm.at[s[0]], s[2], s[5])
        pltpu.async_copy(ex_hbm.at[pl.ds(off, CK)], s[6], s[5])

    def wait_gath1(p):
        s = bufs[p]
        pltpu.make_async_copy(xl_hbm.at[s[0]], s[2], s[5]).wait()
        pltpu.make_async_copy(ex_hbm.at[pl.ds(0, CK)], s[6], s[5]).wait()

    def wait_ex(p):
        s = bufs[p]
        pltpu.make_async_copy(s[6], ex_hbm.at[pl.ds(0, CK)], s[7]).wait()

    def bucket_body(t, _):
        b = w + NW * t

        @pl.when(b < NB)
        def _():
            st0 = _sget(s0_v, b)
            n0 = _sget(n0_v, b)
            st1 = _sget(s1_v, b) + SPCAP
            n1 = _sget(n1_v, b)
            nck0 = (n0 + CK - 1) >> 7
            nck = nck0 + ((n1 + CK - 1) >> 7)
            nbase = b * BN

            def cce(i):
                return jnp.minimum(CK, jnp.where(i < nck0, n0 - i * CK,
                                                 n1 - (i - nck0) * CK))

            def zacc(r, _):
                row = acc.at[r]
                for k in range(5):
                    row[pl.ds(16 * k, 16)] = zf
                return 0

            lax.fori_loop(0, BN, zacc, 0)
            for kk in range(BN // 16):
                den[pl.ds(16 * kk, 16)] = zf
                den1[pl.ds(16 * kk, 16)] = zf
                den2[pl.ds(16 * kk, 16)] = zf
                den3[pl.ds(16 * kk, 16)] = zf

            def coff(i):
                return pl.multiple_of(
                    jnp.where(i < nck0, st0 + i * CK,
                              st1 + (i - nck0) * CK), CK)

            @pl.when(nck > 0)
            def _():
                issue_idx(0, coff(0))
                wait_idx(0)
                issue_gath2(0)

                @pl.when(nck > 1)
                def _():
                    issue_idx(1, coff(1))

            def p1pair(i2, _):
                for sub in range(2):
                    i = i2 * 2 + sub
                    par = sub
                    nxt = 1 - sub
                    s = bufs[par]

                    @pl.when(i < nck)
                    def _():
                        wait_gath2(par)
                        ce = cce(i)
                        sdidx, sxl, sxr, sex = s[1], s[2], s[3], s[6]

                        def eb(j, _):
                            lrow = sxl.at[j]
                            rrow = sxr.at[j]
                            av = zf
                            for k in range(5):
                                sv = (lrow[pl.ds(16 * k, 16)]
                                      + rrow[pl.ds(16 * k, 16)])
                                lr = jnp.maximum(sv, NEG * sv)
                                av = av + attk[k] * lr
                            for sh in (8, 4, 2, 1):
                                av = av + av[jnp.arange(16) ^ sh]
                            albuf[pl.ds(j, 1)] = av[0:1]
                            return 0

                        lax.fori_loop(0, ce, eb, 0)

                        @pl.when(i >= 2)
                        def _():
                            wait_ex(par)
                        for kk in range(CK // 16):
                            sex[pl.ds(16 * kk, 16)] = jnp.exp(
                                albuf[pl.ds(16 * kk, 16)])
                        pltpu.async_copy(sex, ex_hbm.at[pl.ds(coff(i), CK)],
                                         s[7])

                        @pl.when(i + 1 < nck)
                        def _():
                            wait_idx(nxt)
                            issue_gath2(nxt)

                        dens = (den, den1, den2, den3)

                        def dbg(g, _):
                            dv = sdidx[pl.ds(g * 16, 16)] - nbase
                            for jj in range(16):
                                dl = dv[jj]
                                dq = dens[jj % 4]
                                dq[pl.ds(dl, 1)] = (dq[pl.ds(dl, 1)]
                                                    + sex[pl.ds(g * 16 + jj,
                                                                1)])
                            return 0

                        lax.fori_loop(0, ce >> 4, dbg, 0)

                        def db(j, _):
                            dl = _sget(sdidx, j) - nbase
                            den[pl.ds(dl, 1)] = (den[pl.ds(dl, 1)]
                                                 + sex[pl.ds(j, 1)])
                            return 0

                        lax.fori_loop((ce >> 4) << 4, ce, db, 0)

                        @pl.when(i + 2 < nck)
                        def _():
                            issue_idx(par, coff(i + 2))
                return 0

            lax.fori_loop(0, (nck + 1) >> 1, p1pair, 0)

            @pl.when(nck >= 1)
            def _():
                wait_ex(0)

            @pl.when(nck >= 2)
            def _():
                wait_ex(1)

            one = jnp.full((16,), 1.0, jnp.float32)
            for kk in range(BN // 16):
                sl = pl.ds(16 * kk, 16)
                den[sl] = one / (den[sl] + den1[sl] + den2[sl] + den3[sl]
                                 + 1e-16)

            @pl.when(nck > 0)
            def _():
                issue_idx(0, coff(0))
                wait_idx(0)
                issue_gath1(0, coff(0))

                @pl.when(nck > 1)
                def _():
                    issue_idx(1, coff(1))

            def p2pair(i2, _):
                for sub in range(2):
                    i = i2 * 2 + sub
                    par = sub
                    nxt = 1 - sub
                    s = bufs[par]

                    @pl.when(i < nck)
                    def _():
                        wait_gath1(par)

                        @pl.when(i + 1 < nck)
                        def _():
                            wait_idx(nxt)
                            issue_gath1(nxt, coff(i + 1))
                        ce = cce(i)
                        sdidx, sxl, sex = s[1], s[2], s[6]

                        def eb2g(g, _):
                            dv = sdidx[pl.ds(g * 16, 16)] - nbase
                            for jj in range(16):
                                j = g * 16 + jj
                                dl = dv[jj]
                                c1 = sex[pl.ds(j, 1)] * den[pl.ds(dl, 1)]
                                c = c1[0]
                                arow = acc.at[dl]
                                lrow = sxl.at[j]
                                for k in range(5):
                                    sl = pl.ds(16 * k, 16)
                                    arow[sl] = arow[sl] + c * lrow[sl]
                            return 0

                        lax.fori_loop(0, ce >> 4, eb2g, 0)

                        def eb2(j, _):
                            dl = _sget(sdidx, j) - nbase
                            c1 = sex[pl.ds(j, 1)] * den[pl.ds(dl, 1)]
                            c = c1[0]
                            arow = acc.at[dl]
                            lrow = sxl.at[j]
                            for k in range(5):
                                sl = pl.ds(16 * k, 16)
                                arow[sl] = arow[sl] + c * lrow[sl]
                            return 0

                        lax.fori_loop((ce >> 4) << 4, ce, eb2, 0)

                        @pl.when(i + 2 < nck)
                        def _():
                            issue_idx(par, coff(i + 2))
                return 0

            lax.fori_loop(0, (nck + 1) >> 1, p2pair, 0)
            pltpu.sync_copy(acc, o_hbm.at[pl.ds(nbase, BN)])

        return 0

    lax.fori_loop(0, TMAX, bucket_body, 0)


_main = pl.kernel(
    _main_body,
    compiler_params=pltpu.CompilerParams(use_tc_tiling_on_sc=False),
    out_type=(
        jax.ShapeDtypeStruct((NB * BN, DP), jnp.float32),
        jax.ShapeDtypeStruct((EPAD,), jnp.float32),
    ),
    mesh=_mesh,
    name="sc_main",
    scratch_types=[
        pltpu.VMEM((NBP,), jnp.int32),
        pltpu.VMEM((NBP,), jnp.int32),
        pltpu.VMEM((NBP,), jnp.int32),
        pltpu.VMEM((NBP,), jnp.int32),
        pltpu.VMEM((DP,), jnp.float32),
        pltpu.VMEM((CK,), jnp.int32),
        pltpu.VMEM((CK,), jnp.int32),
        pltpu.VMEM((CK,), jnp.int32),
        pltpu.VMEM((CK,), jnp.int32),
        pltpu.VMEM((CK, DP), jnp.float32),
        pltpu.VMEM((CK, DP), jnp.float32),
        pltpu.VMEM((CK, DP), jnp.float32),
        pltpu.VMEM((CK, DP), jnp.float32),
        pltpu.VMEM((BN, DP), jnp.float32),
        pltpu.VMEM((BN,), jnp.float32),
        pltpu.VMEM((BN,), jnp.float32),
        pltpu.VMEM((BN,), jnp.float32),
        pltpu.VMEM((BN,), jnp.float32),
        pltpu.VMEM((CK,), jnp.float32),
        pltpu.VMEM((CK,), jnp.float32),
        pltpu.VMEM((CK,), jnp.float32),
        pltpu.SemaphoreType.DMA,
        pltpu.SemaphoreType.DMA,
        pltpu.SemaphoreType.DMA,
        pltpu.SemaphoreType.DMA,
        pltpu.SemaphoreType.DMA,
        pltpu.SemaphoreType.DMA,
    ],
)


def _conv_sc(xl, xr, att80, ei):
    srcp = jnp.pad(ei[0], (0, CK))
    dstp = jnp.pad(ei[1], (0, CK))
    hist = _hist(dstp)
    srcs, dsts, a0, c0, a1, c1 = _scat(srcp, dstp, hist)
    o, _ = _main(xl, xr, att80, srcs, dsts, a0, c0, a1, c1)
    return o


def _fin_body(of_ref, ob_ref, xf_ref, xb_ref, bf, bb, out_ref):
    f = jnp.maximum(of_ref[:, :D] + bf[:, :D] + xf_ref[...], 0.0)
    g = jnp.maximum(ob_ref[:, :D] + bb[:, :D] + xb_ref[...], 0.0)
    out_ref[...] = jnp.concatenate([f, g], axis=-1)


def _final(of, ob, xf, xb, bf, bb):
    blk = 2000
    grid = N // blk
    ospec = pl.BlockSpec((blk, DP), lambda i: (i, 0))
    xspec = pl.BlockSpec((blk, D), lambda i: (i, 0))
    bspec = pl.BlockSpec((1, DP), lambda i: (0, 0))
    return pl.pallas_call(
        _fin_body,
        grid=grid,
        in_specs=[ospec, ospec, xspec, xspec, bspec, bspec],
        out_specs=pl.BlockSpec((blk, 2 * D), lambda i: (i, 0)),
        out_shape=jax.ShapeDtypeStruct((N, 2 * D), jnp.float32),
    )(of, ob, xf, xb, bf, bb)


def kernel(x_fwd, edge_index_fwd, x_bwd, edge_index_bwd,
           Wl_f, bl_f, Wr_f, br_f, att_f, bias_f,
           Wl_b, bl_b, Wr_b, br_b, att_b, bias_b):
    padw = lambda m: jnp.pad(m, ((0, 0), (0, DP - D)))
    padv = lambda v: jnp.pad(v, (0, DP - D)).reshape(1, DP)
    xlf, xrf, xlb, xrb = _linear(
        x_fwd, x_bwd,
        padw(Wl_f), padv(bl_f), padw(Wr_f), padv(br_f),
        padw(Wl_b), padv(bl_b), padw(Wr_b), padv(br_b))
    of = _conv_sc(xlf, xrf, jnp.pad(att_f, (0, DP - D)), edge_index_fwd)
    ob = _conv_sc(xlb, xrb, jnp.pad(att_b, (0, DP - D)), edge_index_bwd)
    return _final(of, ob, x_fwd, x_bwd, padv(bias_f), padv(bias_b))


# speedup vs baseline: 8.4811x; 1.1493x over previous
import jax
import jax.numpy as jnp
from jax import lax
from jax.experimental import pallas as pl
from jax.experimental.pallas import tpu as pltpu
from jax.experimental.pallas import tpu_sc as plsc

N = 50000
D = 73
DP = 80
E = 800000
NEG = 0.2
BSH = 8
BN = 256
NB = (N + BN - 1) // BN
NBP = 256
NW = 32
EW = E // NW
CK = 128
NCK_W = (EW + CK - 1) // CK
SPCAP = E // 2 + CK * NB + CK
EPAD = 2 * SPCAP + CK
TMAX = (NB + NW - 1) // NW

_mesh = plsc.VectorSubcoreMesh(core_axis_name="c", subcore_axis_name="s")


def _wid():
    return lax.axis_index("s") * 2 + lax.axis_index("c")


def _sget(ref, i):
    return ref[pl.ds(i, 1)][0]


def _lin_body(xf_ref, xb_ref, wlf, blf, wrf, brf, wlb, blb, wrb, brb,
              xlf_ref, xrf_ref, xlb_ref, xrb_ref):
    xf = xf_ref[...]
    xb = xb_ref[...]
    xlf_ref[...] = jnp.dot(xf, wlf[...], preferred_element_type=jnp.float32) + blf[...]
    xrf_ref[...] = jnp.dot(xf, wrf[...], preferred_element_type=jnp.float32) + brf[...]
    xlb_ref[...] = jnp.dot(xb, wlb[...], preferred_element_type=jnp.float32) + blb[...]
    xrb_ref[...] = jnp.dot(xb, wrb[...], preferred_element_type=jnp.float32) + brb[...]


def _linear(xf, xb, wlf, blf, wrf, brf, wlb, blb, wrb, brb):
    blk = 2000
    grid = N // blk
    xspec = pl.BlockSpec((blk, D), lambda i: (i, 0))
    wspec = pl.BlockSpec((D, DP), lambda i: (0, 0))
    bspec = pl.BlockSpec((1, DP), lambda i: (0, 0))
    ospec = pl.BlockSpec((blk, DP), lambda i: (i, 0))
    return pl.pallas_call(
        _lin_body,
        grid=grid,
        in_specs=[xspec, xspec, wspec, bspec, wspec, bspec,
                  wspec, bspec, wspec, bspec],
        out_specs=[ospec, ospec, ospec, ospec],
        out_shape=[jax.ShapeDtypeStruct((N, DP), jnp.float32)] * 4,
    )(xf, xb, wlf, blf, wrf, brf, wlb, blb, wrb, brb)


def _hist_body(dst_hbm, hist_hbm, dbuf, hv, hs):
    w = _wid()
    pltpu.sync_copy(dst_hbm.at[pl.ds(pl.multiple_of(w * EW, 8), EW)], dbuf)

    def zb(b, _):
        hs[b] = 0
        return 0

    lax.fori_loop(0, NBP, zb, 0)

    def egroup(g, _):
        dv = dbuf[pl.ds(g * 16, 16)] >> BSH
        for jj in range(16):
            b = dv[jj]
            hs[b] = hs[b] + 1
        return 0

    lax.fori_loop(0, EW >> 4, egroup, 0)

    def body(e, _):
        b = _sget(dbuf, e) >> BSH
        hs[b] = hs[b] + 1
        return 0

    lax.fori_loop((EW >> 4) << 4, EW, body, 0)

    def cp(b, _):
        hv[pl.ds(b, 1)] = jnp.reshape(hs[b], (1,))
        return 0

    lax.fori_loop(0, NBP, cp, 0)
    pltpu.sync_copy(hv, hist_hbm.at[w])


_hist = pl.kernel(
    _hist_body,
    out_type=jax.ShapeDtypeStruct((NW, NBP), jnp.int32),
    mesh=_mesh,
    name="sc_hist",
    scratch_types=[
        pltpu.VMEM((EW,), jnp.int32),
        pltpu.VMEM((NBP,), jnp.int32),
        pltpu.SMEM((NBP,), jnp.int32),
    ],
)


def _scat_body(src_hbm, dst_hbm, hist_hbm, srcs_hbm, dsts_hbm,
               a0_hbm, c0_hbm, a1_hbm, c1_hbm,
               hall, cur, a0_v, c0_v, a1_v, c1_v,
               sbuf, dbuf, posb, sbuf1, dbuf1, posb1, sbuf2, dbuf2, posb2,
               srcs_sp, dsts_sp, curs,
               semr, semw, semr1, semw1, semr2, semw2):
    w = _wid()
    h = w & 1
    sid = lax.axis_index("s")
    pltpu.sync_copy(hist_hbm, hall)
    zi = jnp.zeros((16,), jnp.int32)

    for kk in range(NBP // 16):
        sl = pl.ds(16 * kk, 16)

        def ws0(k, c):
            return c + hall[2 * k, sl]

        def ws1(k, c):
            return c + hall[2 * k + 1, sl]

        def wsm(k, c):
            return c + hall[2 * k + h, sl]

        c0_v[sl] = lax.fori_loop(0, 16, ws0, zi)
        c1_v[sl] = lax.fori_loop(0, 16, ws1, zi)
        cur[sl] = lax.fori_loop(0, sid, wsm, zi)

    def bloop0(b, a):
        cs = _sget(c0_v, b)
        a0_v[pl.ds(b, 1)] = jnp.reshape(a, (1,))
        return (a + cs + CK - 1) & (-CK)

    lax.fori_loop(0, NB, bloop0, 0)

    def bloop1(b, a):
        cs = _sget(c1_v, b)
        a1_v[pl.ds(b, 1)] = jnp.reshape(a, (1,))
        return (a + cs + CK - 1) & (-CK)

    lax.fori_loop(0, NB, bloop1, 0)

    def cinit(b, _):
        ab = jnp.where(h == 0, _sget(a0_v, b), _sget(a1_v, b))
        curs[b] = ab + _sget(cur, b)
        return 0

    lax.fori_loop(0, NB, cinit, 0)

    base = w * EW
    bufs = ((sbuf, dbuf, posb, semr, semw),
            (sbuf1, dbuf1, posb1, semr1, semw1),
            (sbuf2, dbuf2, posb2, semr2, semw2))

    def issue_read(p, i):
        off = pl.multiple_of(base + i * CK, 8)
        s = bufs[p]
        pltpu.async_copy(src_hbm.at[pl.ds(off, CK)], s[0], s[3])
        pltpu.async_copy(dst_hbm.at[pl.ds(off, CK)], s[1], s[3])

    def wait_read(p):
        s = bufs[p]
        pltpu.make_async_copy(src_hbm.at[pl.ds(0, CK)], s[0], s[3]).wait()
        pltpu.make_async_copy(dst_hbm.at[pl.ds(0, CK)], s[1], s[3]).wait()

    def issue_scat(p):
        s = bufs[p]
        pltpu.async_copy(s[0], srcs_sp.at[s[2]], s[4])
        pltpu.async_copy(s[1], dsts_sp.at[s[2]], s[4])

    def wait_scat(p):
        s = bufs[p]
        pltpu.make_async_copy(s[0], srcs_sp.at[s[2]], s[4]).wait()
        pltpu.make_async_copy(s[1], dsts_sp.at[s[2]], s[4]).wait()

    issue_read(0, 0)

    def chunk3(i3, _):
        for sub in range(3):
            i = i3 * 3 + sub
            par = sub
            nxt = (sub + 1) % 3

            @pl.when(i < NCK_W)
            def _():
                @pl.when(i + 1 < NCK_W)
                def _():
                    @pl.when(i >= 2)
                    def _():
                        wait_scat(nxt)
                    issue_read(nxt, i + 1)
                wait_read(par)
                s = bufs[par]
                sdbuf, sposb = s[1], s[2]
                ce = jnp.minimum(CK, EW - i * CK)

                def egroup(g, _):
                    dv = sdbuf[pl.ds(g * 16, 16)] >> BSH
                    for jj in range(16):
                        b = dv[jj]
                        p = curs[b]
                        curs[b] = p + 1
                        sposb[pl.ds(g * 16 + jj, 1)] = jnp.reshape(p, (1,))
                    return 0

                lax.fori_loop(0, ce >> 4, egroup, 0)

                def ebody(j, _):
                    b = _sget(sdbuf, j) >> BSH
                    p = curs[b]
                    curs[b] = p + 1
                    sposb[pl.ds(j, 1)] = jnp.reshape(p, (1,))
                    return 0

                lax.fori_loop((ce >> 4) << 4, ce, ebody, 0)

                def tbody(j, _):
                    sposb[pl.ds(j, 1)] = jnp.reshape(SPCAP + j, (1,))
                    return 0

                lax.fori_loop(ce, CK, tbody, 0)
                issue_scat(par)
        return 0

    lax.fori_loop(0, (NCK_W + 2) // 3, chunk3, 0)
    for p in range(3):
        wait_scat(p)
    plsc.subcore_barrier()

    @pl.when(sid == 0)
    def _():
        off = pl.multiple_of(h * SPCAP, 8)
        pltpu.async_copy(srcs_sp.at[pl.ds(0, SPCAP)],
                         srcs_hbm.at[pl.ds(off, SPCAP)], semr)
        pltpu.async_copy(dsts_sp.at[pl.ds(0, SPCAP)],
                         dsts_hbm.at[pl.ds(off, SPCAP)], semr)
        pltpu.make_async_copy(srcs_sp.at[pl.ds(0, SPCAP)],
                              srcs_hbm.at[pl.ds(off, SPCAP)], semr).wait()
        pltpu.make_async_copy(dsts_sp.at[pl.ds(0, SPCAP)],
                              dsts_hbm.at[pl.ds(off, SPCAP)], semr).wait()

    @pl.when(w == 0)
    def _():
        pltpu.sync_copy(a0_v, a0_hbm)
        pltpu.sync_copy(c0_v, c0_hbm)
        pltpu.sync_copy(a1_v, a1_hbm)
        pltpu.sync_copy(c1_v, c1_hbm)


_scat = pl.kernel(
    _scat_body,
    out_type=(
        jax.ShapeDtypeStruct((EPAD,), jnp.int32),
        jax.ShapeDtypeStruct((EPAD,), jnp.int32),
        jax.ShapeDtypeStruct((NBP,), jnp.int32),
        jax.ShapeDtypeStruct((NBP,), jnp.int32),
        jax.ShapeDtypeStruct((NBP,), jnp.int32),
        jax.ShapeDtypeStruct((NBP,), jnp.int32),
    ),
    mesh=_mesh,
    name="sc_scat",
    scratch_types=[
        pltpu.VMEM((NW, NBP), jnp.int32),
        pltpu.VMEM((NBP,), jnp.int32),
        pltpu.VMEM((NBP,), jnp.int32),
        pltpu.VMEM((NBP,), jnp.int32),
        pltpu.VMEM((NBP,), jnp.int32),
        pltpu.VMEM((NBP,), jnp.int32),
        pltpu.VMEM((CK,), jnp.int32),
        pltpu.VMEM((CK,), jnp.int32),
        pltpu.VMEM((CK,), jnp.int32),
        pltpu.VMEM((CK,), jnp.int32),
        pltpu.VMEM((CK,), jnp.int32),
        pltpu.VMEM((CK,), jnp.int32),
        pltpu.VMEM((CK,), jnp.int32),
        pltpu.VMEM((CK,), jnp.int32),
        pltpu.VMEM((CK,), jnp.int32),
        pltpu.VMEM_SHARED((SPCAP + CK,), jnp.int32),
        pltpu.VMEM_SHARED((SPCAP + CK,), jnp.int32),
        pltpu.SMEM((NBP,), jnp.int32),
        pltpu.SemaphoreType.DMA,
        pltpu.SemaphoreType.DMA,
        pltpu.SemaphoreType.DMA,
        pltpu.SemaphoreType.DMA,
        pltpu.SemaphoreType.DMA,
        pltpu.SemaphoreType.DMA,
    ],
)


def _main_body(xl_hbm, xr_hbm, att_hbm, srcs_hbm, dsts_hbm, a0_hbm,
               c0_hbm, a1_hbm, c1_hbm, o_hbm, ex_hbm,
               s0_v, n0_v, s1_v, n1_v, attv,
               sidx0, didx0, sidx1, didx1, xlrow0, xrrow0, xlrow1, xrrow1,
               acc, den, den1, den2, den3, albuf, exbuf0, exbuf1,
               semi0, semi1, semg0, semg1, semx0, semx1):
    w = _wid()
    pltpu.sync_copy(a0_hbm, s0_v)
    pltpu.sync_copy(c0_hbm, n0_v)
    pltpu.sync_copy(a1_hbm, s1_v)
    pltpu.sync_copy(c1_hbm, n1_v)
    pltpu.sync_copy(att_hbm, attv)
    attk = [attv[pl.ds(16 * k, 16)] for k in range(5)]
    zf = jnp.zeros((16,), jnp.float32)
    zi = jnp.zeros((16,), jnp.int32)
    bufs = ((sidx0, didx0, xlrow0, xrrow0, semi0, semg0, exbuf0, semx0),
            (sidx1, didx1, xlrow1, xrrow1, semi1, semg1, exbuf1, semx1))

    def clamp_idx(ref, hi):
        for kk in range(CK // 16):
            v = ref[pl.ds(16 * kk, 16)]
            ref[pl.ds(16 * kk, 16)] = jnp.minimum(jnp.maximum(v, zi), hi)

    def issue_idx(p, off):
        s = bufs[p]
        pltpu.async_copy(srcs_hbm.at[pl.ds(off, CK)], s[0], s[4])
        pltpu.async_copy(dsts_hbm.at[pl.ds(off, CK)], s[1], s[4])

    def wait_idx(p):
        s = bufs[p]
        pltpu.make_async_copy(srcs_hbm.at[pl.ds(0, CK)], s[0], s[4]).wait()
        pltpu.make_async_copy(dsts_hbm.at[pl.ds(0, CK)], s[1], s[4]).wait()

    def issue_gath2(p):
        s = bufs[p]
        clamp_idx(s[0], N - 1)
        clamp_idx(s[1], N - 1)
        pltpu.async_copy(xl_hbm.at[s[0]], s[2], s[5])
        pltpu.async_copy(xr_hbm.at[s[1]], s[3], s[5])

    def wait_gath2(p):
        s = bufs[p]
        pltpu.make_async_copy(xl_hbm.at[s[0]], s[2], s[5]).wait()
        pltpu.make_async_copy(xr_hbm.at[s[1]], s[3], s[5]).wait()

    def issue_gath1(p, off):
        s = bufs[p]
        clamp_idx(s[0], N - 1)
        pltpu.async_copy(xl_hbm.at[s[0]], s[2], s[5])
        pltpu.async_copy(ex_hbm.at[pl.ds(off, CK)], s[6], s[5])

    def wait_gath1(p):
        s = bufs[p]
        pltpu.make_async_copy(xl_hbm.at[s[0]], s[2], s[5]).wait()
        pltpu.make_async_copy(ex_hbm.at[pl.ds(0, CK)], s[6], s[5]).wait()

    def wait_ex(p):
        s = bufs[p]
        pltpu.make_async_copy(s[6], ex_hbm.at[pl.ds(0, CK)], s[7]).wait()

    def bucket_body(t, _):
        b = w + NW * t

        @pl.when(b < NB)
        def _():
            st0 = _sget(s0_v, b)
            n0 = _sget(n0_v, b)
            st1 = _sget(s1_v, b) + SPCAP
            n1 = _sget(n1_v, b)
            nck0 = (n0 + CK - 1) >> 7
            nck = nck0 + ((n1 + CK - 1) >> 7)
            nbase = b * BN

            def cce(i):
                return jnp.minimum(CK, jnp.where(i < nck0, n0 - i * CK,
                                                 n1 - (i - nck0) * CK))

            def zacc(r, _):
                row = acc.at[r]
                for k in range(5):
                    row[pl.ds(16 * k, 16)] = zf
                return 0

            lax.fori_loop(0, BN, zacc, 0)
            for kk in range(BN // 16):
                den[pl.ds(16 * kk, 16)] = zf
                den1[pl.ds(16 * kk, 16)] = zf
                den2[pl.ds(16 * kk, 16)] = zf
                den3[pl.ds(16 * kk, 16)] = zf

            def coff(i):
                return pl.multiple_of(
                    jnp.where(i < nck0, st0 + i * CK,
                              st1 + (i - nck0) * CK), CK)

            @pl.when(nck > 0)
            def _():
                issue_idx(0, coff(0))
                wait_idx(0)
                issue_gath2(0)

                @pl.when(nck > 1)
                def _():
                    issue_idx(1, coff(1))

            def p1pair(i2, _):
                for sub in range(2):
                    i = i2 * 2 + sub
                    par = sub
                    nxt = 1 - sub
                    s = bufs[par]

                    @pl.when(i < nck)
                    def _():
                        wait_gath2(par)
                        ce = cce(i)
                        sdidx, sxl, sxr, sex = s[1], s[2], s[3], s[6]

                        @pl.when(i >= 2)
                        def _():
                            wait_ex(par)
                        lane = jnp.arange(16)

                        def pg(g, _):
                            vs = []
                            for jj in range(16):
                                lrow = sxl.at[g * 16 + jj]
                                rrow = sxr.at[g * 16 + jj]
                                av = zf
                                for k in range(5):
                                    sv = (lrow[pl.ds(16 * k, 16)]
                                          + rrow[pl.ds(16 * k, 16)])
                                    lr = jnp.maximum(sv, NEG * sv)
                                    av = av + attk[k] * lr
                                vs.append(av)
                            for sh in (8, 4, 2, 1):
                                m = (lane % (2 * sh)) < sh
                                half = len(vs) // 2
                                vs = [jnp.where(m, vs[i2], vs[i2 + half][lane ^ sh])
                                      + jnp.where(m, vs[i2][lane ^ sh],
                                                  vs[i2 + half])
                                      for i2 in range(half)]
                            sex[pl.ds(pl.multiple_of(g * 16, 16), 16)] = (
                                jnp.exp(vs[0]))
                            return 0

                        lax.fori_loop(0, CK // 16, pg, 0)
                        pltpu.async_copy(sex, ex_hbm.at[pl.ds(coff(i), CK)],
                                         s[7])

                        @pl.when(i + 1 < nck)
                        def _():
                            wait_idx(nxt)
                            issue_gath2(nxt)

                        dens = (den, den1, den2, den3)

                        def dbg(g, _):
                            dv = sdidx[pl.ds(g * 16, 16)] - nbase
                            for jj in range(16):
                                dl = dv[jj]
                                dq = dens[jj % 4]
                                dq[pl.ds(dl, 1)] = (dq[pl.ds(dl, 1)]
                                                    + sex[pl.ds(g * 16 + jj,
                                                                1)])
                            return 0

                        lax.fori_loop(0, ce >> 4, dbg, 0)

                        def db(j, _):
                            dl = _sget(sdidx, j) - nbase
                            den[pl.ds(dl, 1)] = (den[pl.ds(dl, 1)]
                                                 + sex[pl.ds(j, 1)])
                            return 0

                        lax.fori_loop((ce >> 4) << 4, ce, db, 0)

                        @pl.when(i + 2 < nck)
                        def _():
                            issue_idx(par, coff(i + 2))
                return 0

            lax.fori_loop(0, (nck + 1) >> 1, p1pair, 0)

            @pl.when(nck >= 1)
            def _():
                wait_ex(0)

            @pl.when(nck >= 2)
            def _():
                wait_ex(1)

            one = jnp.full((16,), 1.0, jnp.float32)
            for kk in range(BN // 16):
                sl = pl.ds(16 * kk, 16)
                den[sl] = one / (den[sl] + den1[sl] + den2[sl] + den3[sl]
                                 + 1e-16)

            @pl.when(nck > 0)
            def _():
                issue_idx(0, coff(0))
                wait_idx(0)
                issue_gath1(0, coff(0))

                @pl.when(nck > 1)
                def _():
                    issue_idx(1, coff(1))

            def p2pair(i2, _):
                for sub in range(2):
                    i = i2 * 2 + sub
                    par = sub
                    nxt = 1 - sub
                    s = bufs[par]

                    @pl.when(i < nck)
                    def _():
                        wait_gath1(par)

                        @pl.when(i + 1 < nck)
                        def _():
                            wait_idx(nxt)
                            issue_gath1(nxt, coff(i + 1))
                        ce = cce(i)
                        sdidx, sxl, sex = s[1], s[2], s[6]

                        def eb2g(g, _):
                            dv = sdidx[pl.ds(g * 16, 16)] - nbase
                            for jj in range(16):
                                j = g * 16 + jj
                                dl = dv[jj]
                                c1 = sex[pl.ds(j, 1)] * den[pl.ds(dl, 1)]
                                c = c1[0]
                                arow = acc.at[dl]
                                lrow = sxl.at[j]
                                for k in range(5):
                                    sl = pl.ds(16 * k, 16)
                                    arow[sl] = arow[sl] + c * lrow[sl]
                            return 0

                        lax.fori_loop(0, ce >> 4, eb2g, 0)

                        def eb2(j, _):
                            dl = _sget(sdidx, j) - nbase
                            c1 = sex[pl.ds(j, 1)] * den[pl.ds(dl, 1)]
                            c = c1[0]
                            arow = acc.at[dl]
                            lrow = sxl.at[j]
                            for k in range(5):
                                sl = pl.ds(16 * k, 16)
                                arow[sl] = arow[sl] + c * lrow[sl]
                            return 0

                        lax.fori_loop((ce >> 4) << 4, ce, eb2, 0)

                        @pl.when(i + 2 < nck)
                        def _():
                            issue_idx(par, coff(i + 2))
                return 0

            lax.fori_loop(0, (nck + 1) >> 1, p2pair, 0)
            pltpu.sync_copy(acc, o_hbm.at[pl.ds(nbase, BN)])

        return 0

    lax.fori_loop(0, TMAX, bucket_body, 0)


_main = pl.kernel(
    _main_body,
    compiler_params=pltpu.CompilerParams(use_tc_tiling_on_sc=False),
    out_type=(
        jax.ShapeDtypeStruct((NB * BN, DP), jnp.float32),
        jax.ShapeDtypeStruct((EPAD,), jnp.float32),
    ),
    mesh=_mesh,
    name="sc_main",
    scratch_types=[
        pltpu.VMEM((NBP,), jnp.int32),
        pltpu.VMEM((NBP,), jnp.int32),
        pltpu.VMEM((NBP,), jnp.int32),
        pltpu.VMEM((NBP,), jnp.int32),
        pltpu.VMEM((DP,), jnp.float32),
        pltpu.VMEM((CK,), jnp.int32),
        pltpu.VMEM((CK,), jnp.int32),
        pltpu.VMEM((CK,), jnp.int32),
        pltpu.VMEM((CK,), jnp.int32),
        pltpu.VMEM((CK, DP), jnp.float32),
        pltpu.VMEM((CK, DP), jnp.float32),
        pltpu.VMEM((CK, DP), jnp.float32),
        pltpu.VMEM((CK, DP), jnp.float32),
        pltpu.VMEM((BN, DP), jnp.float32),
        pltpu.VMEM((BN,), jnp.float32),
        pltpu.VMEM((BN,), jnp.float32),
        pltpu.VMEM((BN,), jnp.float32),
        pltpu.VMEM((BN,), jnp.float32),
        pltpu.VMEM((CK,), jnp.float32),
        pltpu.VMEM((CK,), jnp.float32),
        pltpu.VMEM((CK,), jnp.float32),
        pltpu.SemaphoreType.DMA,
        pltpu.SemaphoreType.DMA,
        pltpu.SemaphoreType.DMA,
        pltpu.SemaphoreType.DMA,
        pltpu.SemaphoreType.DMA,
        pltpu.SemaphoreType.DMA,
    ],
)


def _conv_sc(xl, xr, att80, ei):
    srcp = jnp.pad(ei[0], (0, CK))
    dstp = jnp.pad(ei[1], (0, CK))
    hist = _hist(dstp)
    srcs, dsts, a0, c0, a1, c1 = _scat(srcp, dstp, hist)
    o, _ = _main(xl, xr, att80, srcs, dsts, a0, c0, a1, c1)
    return o


def _fin_body(of_ref, ob_ref, xf_ref, xb_ref, bf, bb, out_ref):
    f = jnp.maximum(of_ref[:, :D] + bf[:, :D] + xf_ref[...], 0.0)
    g = jnp.maximum(ob_ref[:, :D] + bb[:, :D] + xb_ref[...], 0.0)
    out_ref[...] = jnp.concatenate([f, g], axis=-1)


def _final(of, ob, xf, xb, bf, bb):
    blk = 2000
    grid = N // blk
    ospec = pl.BlockSpec((blk, DP), lambda i: (i, 0))
    xspec = pl.BlockSpec((blk, D), lambda i: (i, 0))
    bspec = pl.BlockSpec((1, DP), lambda i: (0, 0))
    return pl.pallas_call(
        _fin_body,
        grid=grid,
        in_specs=[ospec, ospec, xspec, xspec, bspec, bspec],
        out_specs=pl.BlockSpec((blk, 2 * D), lambda i: (i, 0)),
        out_shape=jax.ShapeDtypeStruct((N, 2 * D), jnp.float32),
    )(of, ob, xf, xb, bf, bb)


def kernel(x_fwd, edge_index_fwd, x_bwd, edge_index_bwd,
           Wl_f, bl_f, Wr_f, br_f, att_f, bias_f,
           Wl_b, bl_b, Wr_b, br_b, att_b, bias_b):
    padw = lambda m: jnp.pad(m, ((0, 0), (0, DP - D)))
    padv = lambda v: jnp.pad(v, (0, DP - D)).reshape(1, DP)
    xlf, xrf, xlb, xrb = _linear(
        x_fwd, x_bwd,
        padw(Wl_f), padv(bl_f), padw(Wr_f), padv(br_f),
        padw(Wl_b), padv(bl_b), padw(Wr_b), padv(br_b))
    of = _conv_sc(xlf, xrf, jnp.pad(att_f, (0, DP - D)), edge_index_fwd)
    ob = _conv_sc(xlb, xrb, jnp.pad(att_b, (0, DP - D)), edge_index_bwd)
    return _final(of, ob, x_fwd, x_bwd, padv(bias_f), padv(bias_b))


# speedup vs baseline: 10.5341x; 1.2421x over previous
import jax
import jax.numpy as jnp
from jax import lax
from jax.experimental import pallas as pl
from jax.experimental.pallas import tpu as pltpu
from jax.experimental.pallas import tpu_sc as plsc

N = 50000
D = 73
DP = 80
E = 800000
NEG = 0.2
BSH = 8
BN = 256
NB = (N + BN - 1) // BN
NBP = 256
NW = 32
EW = E // NW
CK = 128
NCK_W = (EW + CK - 1) // CK
SPCAP = E // 2 + CK * NB + CK
EPAD = 2 * SPCAP + CK
TMAX = (NB + NW - 1) // NW

_mesh = plsc.VectorSubcoreMesh(core_axis_name="c", subcore_axis_name="s")


def _wid():
    return lax.axis_index("s") * 2 + lax.axis_index("c")


def _sget(ref, i):
    return ref[pl.ds(i, 1)][0]


def _lin_body(xf_ref, xb_ref, wlf, blf, wrf, brf, wlb, blb, wrb, brb,
              xlf_ref, xrf_ref, xlb_ref, xrb_ref):
    xf = xf_ref[...]
    xb = xb_ref[...]
    xlf_ref[...] = jnp.dot(xf, wlf[...], preferred_element_type=jnp.float32) + blf[...]
    xrf_ref[...] = jnp.dot(xf, wrf[...], preferred_element_type=jnp.float32) + brf[...]
    xlb_ref[...] = jnp.dot(xb, wlb[...], preferred_element_type=jnp.float32) + blb[...]
    xrb_ref[...] = jnp.dot(xb, wrb[...], preferred_element_type=jnp.float32) + brb[...]


def _linear(xf, xb, wlf, blf, wrf, brf, wlb, blb, wrb, brb):
    blk = 2000
    grid = N // blk
    xspec = pl.BlockSpec((blk, D), lambda i: (i, 0))
    wspec = pl.BlockSpec((D, DP), lambda i: (0, 0))
    bspec = pl.BlockSpec((1, DP), lambda i: (0, 0))
    ospec = pl.BlockSpec((blk, DP), lambda i: (i, 0))
    return pl.pallas_call(
        _lin_body,
        grid=grid,
        in_specs=[xspec, xspec, wspec, bspec, wspec, bspec,
                  wspec, bspec, wspec, bspec],
        out_specs=[ospec, ospec, ospec, ospec],
        out_shape=[jax.ShapeDtypeStruct((N, DP), jnp.float32)] * 4,
    )(xf, xb, wlf, blf, wrf, brf, wlb, blb, wrb, brb)


def _hist_body(dst_hbm, hist_hbm, dbuf, hv, hs):
    w = _wid()
    pltpu.sync_copy(dst_hbm.at[pl.ds(pl.multiple_of(w * EW, 8), EW)], dbuf)

    def zb(b, _):
        hs[b] = 0
        return 0

    lax.fori_loop(0, NBP, zb, 0)

    def egroup(g, _):
        dv = dbuf[pl.ds(g * 16, 16)] >> BSH
        for jj in range(16):
            b = dv[jj]
            hs[b] = hs[b] + 1
        return 0

    lax.fori_loop(0, EW >> 4, egroup, 0)

    def body(e, _):
        b = _sget(dbuf, e) >> BSH
        hs[b] = hs[b] + 1
        return 0

    lax.fori_loop((EW >> 4) << 4, EW, body, 0)

    def cp(b, _):
        hv[pl.ds(b, 1)] = jnp.reshape(hs[b], (1,))
        return 0

    lax.fori_loop(0, NBP, cp, 0)
    pltpu.sync_copy(hv, hist_hbm.at[w])


_hist = pl.kernel(
    _hist_body,
    out_type=jax.ShapeDtypeStruct((NW, NBP), jnp.int32),
    mesh=_mesh,
    name="sc_hist",
    scratch_types=[
        pltpu.VMEM((EW,), jnp.int32),
        pltpu.VMEM((NBP,), jnp.int32),
        pltpu.SMEM((NBP,), jnp.int32),
    ],
)


def _scat_body(src_hbm, dst_hbm, hist_hbm, srcs_hbm, dsts_hbm,
               a0_hbm, c0_hbm, a1_hbm, c1_hbm,
               hall, cur, a0_v, c0_v, a1_v, c1_v,
               sbuf, dbuf, posb, sbuf1, dbuf1, posb1, sbuf2, dbuf2, posb2,
               srcs_sp, dsts_sp, curs,
               semr, semw, semr1, semw1, semr2, semw2):
    w = _wid()
    h = w & 1
    sid = lax.axis_index("s")
    pltpu.sync_copy(hist_hbm, hall)
    zi = jnp.zeros((16,), jnp.int32)

    for kk in range(NBP // 16):
        sl = pl.ds(16 * kk, 16)

        def ws0(k, c):
            return c + hall[2 * k, sl]

        def ws1(k, c):
            return c + hall[2 * k + 1, sl]

        def wsm(k, c):
            return c + hall[2 * k + h, sl]

        c0_v[sl] = lax.fori_loop(0, 16, ws0, zi)
        c1_v[sl] = lax.fori_loop(0, 16, ws1, zi)
        cur[sl] = lax.fori_loop(0, sid, wsm, zi)

    def bloop0(b, a):
        cs = _sget(c0_v, b)
        a0_v[pl.ds(b, 1)] = jnp.reshape(a, (1,))
        return (a + cs + CK - 1) & (-CK)

    lax.fori_loop(0, NB, bloop0, 0)

    def bloop1(b, a):
        cs = _sget(c1_v, b)
        a1_v[pl.ds(b, 1)] = jnp.reshape(a, (1,))
        return (a + cs + CK - 1) & (-CK)

    lax.fori_loop(0, NB, bloop1, 0)

    def cinit(b, _):
        ab = jnp.where(h == 0, _sget(a0_v, b), _sget(a1_v, b))
        curs[b] = ab + _sget(cur, b)
        return 0

    lax.fori_loop(0, NB, cinit, 0)

    base = w * EW
    bufs = ((sbuf, dbuf, posb, semr, semw),
            (sbuf1, dbuf1, posb1, semr1, semw1),
            (sbuf2, dbuf2, posb2, semr2, semw2))

    def issue_read(p, i):
        off = pl.multiple_of(base + i * CK, 8)
        s = bufs[p]
        pltpu.async_copy(src_hbm.at[pl.ds(off, CK)], s[0], s[3])
        pltpu.async_copy(dst_hbm.at[pl.ds(off, CK)], s[1], s[3])

    def wait_read(p):
        s = bufs[p]
        pltpu.make_async_copy(src_hbm.at[pl.ds(0, CK)], s[0], s[3]).wait()
        pltpu.make_async_copy(dst_hbm.at[pl.ds(0, CK)], s[1], s[3]).wait()

    def issue_scat(p):
        s = bufs[p]
        pltpu.async_copy(s[0], srcs_sp.at[s[2]], s[4])
        pltpu.async_copy(s[1], dsts_sp.at[s[2]], s[4])

    def wait_scat(p):
        s = bufs[p]
        pltpu.make_async_copy(s[0], srcs_sp.at[s[2]], s[4]).wait()
        pltpu.make_async_copy(s[1], dsts_sp.at[s[2]], s[4]).wait()

    issue_read(0, 0)

    def chunk3(i3, _):
        for sub in range(3):
            i = i3 * 3 + sub
            par = sub
            nxt = (sub + 1) % 3

            @pl.when(i < NCK_W)
            def _():
                @pl.when(i + 1 < NCK_W)
                def _():
                    @pl.when(i >= 2)
                    def _():
                        wait_scat(nxt)
                    issue_read(nxt, i + 1)
                wait_read(par)
                s = bufs[par]
                sdbuf, sposb = s[1], s[2]
                ce = jnp.minimum(CK, EW - i * CK)

                def egroup(g, _):
                    dv = sdbuf[pl.ds(g * 16, 16)] >> BSH
                    for jj in range(16):
                        b = dv[jj]
                        p = curs[b]
                        curs[b] = p + 1
                        sposb[pl.ds(g * 16 + jj, 1)] = jnp.reshape(p, (1,))
                    return 0

                lax.fori_loop(0, ce >> 4, egroup, 0)

                def ebody(j, _):
                    b = _sget(sdbuf, j) >> BSH
                    p = curs[b]
                    curs[b] = p + 1
                    sposb[pl.ds(j, 1)] = jnp.reshape(p, (1,))
                    return 0

                lax.fori_loop((ce >> 4) << 4, ce, ebody, 0)

                def tbody(j, _):
                    sposb[pl.ds(j, 1)] = jnp.reshape(SPCAP + j, (1,))
                    return 0

                lax.fori_loop(ce, CK, tbody, 0)
                issue_scat(par)
        return 0

    lax.fori_loop(0, (NCK_W + 2) // 3, chunk3, 0)
    for p in range(3):
        wait_scat(p)
    plsc.subcore_barrier()

    @pl.when(sid == 0)
    def _():
        off = pl.multiple_of(h * SPCAP, 8)
        pltpu.async_copy(srcs_sp.at[pl.ds(0, SPCAP)],
                         srcs_hbm.at[pl.ds(off, SPCAP)], semr)
        pltpu.async_copy(dsts_sp.at[pl.ds(0, SPCAP)],
                         dsts_hbm.at[pl.ds(off, SPCAP)], semr)
        pltpu.make_async_copy(srcs_sp.at[pl.ds(0, SPCAP)],
                              srcs_hbm.at[pl.ds(off, SPCAP)], semr).wait()
        pltpu.make_async_copy(dsts_sp.at[pl.ds(0, SPCAP)],
                              dsts_hbm.at[pl.ds(off, SPCAP)], semr).wait()

    @pl.when(w == 0)
    def _():
        pltpu.sync_copy(a0_v, a0_hbm)
        pltpu.sync_copy(c0_v, c0_hbm)
        pltpu.sync_copy(a1_v, a1_hbm)
        pltpu.sync_copy(c1_v, c1_hbm)


_scat = pl.kernel(
    _scat_body,
    out_type=(
        jax.ShapeDtypeStruct((EPAD,), jnp.int32),
        jax.ShapeDtypeStruct((EPAD,), jnp.int32),
        jax.ShapeDtypeStruct((NBP,), jnp.int32),
        jax.ShapeDtypeStruct((NBP,), jnp.int32),
        jax.ShapeDtypeStruct((NBP,), jnp.int32),
        jax.ShapeDtypeStruct((NBP,), jnp.int32),
    ),
    mesh=_mesh,
    name="sc_scat",
    scratch_types=[
        pltpu.VMEM((NW, NBP), jnp.int32),
        pltpu.VMEM((NBP,), jnp.int32),
        pltpu.VMEM((NBP,), jnp.int32),
        pltpu.VMEM((NBP,), jnp.int32),
        pltpu.VMEM((NBP,), jnp.int32),
        pltpu.VMEM((NBP,), jnp.int32),
        pltpu.VMEM((CK,), jnp.int32),
        pltpu.VMEM((CK,), jnp.int32),
        pltpu.VMEM((CK,), jnp.int32),
        pltpu.VMEM((CK,), jnp.int32),
        pltpu.VMEM((CK,), jnp.int32),
        pltpu.VMEM((CK,), jnp.int32),
        pltpu.VMEM((CK,), jnp.int32),
        pltpu.VMEM((CK,), jnp.int32),
        pltpu.VMEM((CK,), jnp.int32),
        pltpu.VMEM_SHARED((SPCAP + CK,), jnp.int32),
        pltpu.VMEM_SHARED((SPCAP + CK,), jnp.int32),
        pltpu.SMEM((NBP,), jnp.int32),
        pltpu.SemaphoreType.DMA,
        pltpu.SemaphoreType.DMA,
        pltpu.SemaphoreType.DMA,
        pltpu.SemaphoreType.DMA,
        pltpu.SemaphoreType.DMA,
        pltpu.SemaphoreType.DMA,
    ],
)


def _main_body(xl_hbm, xr_hbm, att_hbm, srcs_hbm, dsts_hbm, a0_hbm,
               c0_hbm, a1_hbm, c1_hbm, o_hbm,
               s0_v, n0_v, s1_v, n1_v, attv,
               sidx0, didx0, sidx1, didx1, xlrow0, xrrow0, xlrow1, xrrow1,
               acc, den, den1, den2, den3, albuf, exbuf0, exbuf1,
               semi0, semi1, semg0, semg1, semx0, semx1):
    w = _wid()
    pltpu.sync_copy(a0_hbm, s0_v)
    pltpu.sync_copy(c0_hbm, n0_v)
    pltpu.sync_copy(a1_hbm, s1_v)
    pltpu.sync_copy(c1_hbm, n1_v)
    pltpu.sync_copy(att_hbm, attv)
    attk = [attv[pl.ds(16 * k, 16)] for k in range(5)]
    zf = jnp.zeros((16,), jnp.float32)
    zi = jnp.zeros((16,), jnp.int32)
    bufs = ((sidx0, didx0, xlrow0, xrrow0, semi0, semg0, exbuf0, semx0),
            (sidx1, didx1, xlrow1, xrrow1, semi1, semg1, exbuf1, semx1))

    def clamp_idx(ref, hi):
        for kk in range(CK // 16):
            v = ref[pl.ds(16 * kk, 16)]
            ref[pl.ds(16 * kk, 16)] = jnp.minimum(jnp.maximum(v, zi), hi)

    def issue_idx(p, off):
        s = bufs[p]
        pltpu.async_copy(srcs_hbm.at[pl.ds(off, CK)], s[0], s[4])
        pltpu.async_copy(dsts_hbm.at[pl.ds(off, CK)], s[1], s[4])

    def wait_idx(p):
        s = bufs[p]
        pltpu.make_async_copy(srcs_hbm.at[pl.ds(0, CK)], s[0], s[4]).wait()
        pltpu.make_async_copy(dsts_hbm.at[pl.ds(0, CK)], s[1], s[4]).wait()

    def issue_gath2(p):
        s = bufs[p]
        clamp_idx(s[0], N - 1)
        clamp_idx(s[1], N - 1)
        pltpu.async_copy(xl_hbm.at[s[0]], s[2], s[5])
        pltpu.async_copy(xr_hbm.at[s[1]], s[3], s[5])

    def wait_gath2(p):
        s = bufs[p]
        pltpu.make_async_copy(xl_hbm.at[s[0]], s[2], s[5]).wait()
        pltpu.make_async_copy(xr_hbm.at[s[1]], s[3], s[5]).wait()

    def bucket_body(t, _):
        b = w + NW * t

        @pl.when(b < NB)
        def _():
            st0 = _sget(s0_v, b)
            n0 = _sget(n0_v, b)
            st1 = _sget(s1_v, b) + SPCAP
            n1 = _sget(n1_v, b)
            nck0 = (n0 + CK - 1) >> 7
            nck = nck0 + ((n1 + CK - 1) >> 7)
            nbase = b * BN

            def cce(i):
                return jnp.minimum(CK, jnp.where(i < nck0, n0 - i * CK,
                                                 n1 - (i - nck0) * CK))

            def zacc(r, _):
                row = acc.at[r]
                for k in range(5):
                    row[pl.ds(16 * k, 16)] = zf
                return 0

            lax.fori_loop(0, BN, zacc, 0)
            for kk in range(BN // 16):
                den[pl.ds(16 * kk, 16)] = zf
                den1[pl.ds(16 * kk, 16)] = zf
                den2[pl.ds(16 * kk, 16)] = zf
                den3[pl.ds(16 * kk, 16)] = zf

            def coff(i):
                return pl.multiple_of(
                    jnp.where(i < nck0, st0 + i * CK,
                              st1 + (i - nck0) * CK), CK)

            @pl.when(nck > 0)
            def _():
                issue_idx(0, coff(0))
                wait_idx(0)
                issue_gath2(0)

                @pl.when(nck > 1)
                def _():
                    issue_idx(1, coff(1))

            def p1pair(i2, _):
                for sub in range(2):
                    i = i2 * 2 + sub
                    par = sub
                    nxt = 1 - sub
                    s = bufs[par]

                    @pl.when(i < nck)
                    def _():
                        wait_gath2(par)
                        ce = cce(i)
                        sdidx, sxl, sxr, sex = s[1], s[2], s[3], s[6]
                        lane = jnp.arange(16)

                        def pg(g, _):
                            vs = []
                            for jj in range(16):
                                lrow = sxl.at[g * 16 + jj]
                                rrow = sxr.at[g * 16 + jj]
                                av = zf
                                for k in range(5):
                                    sv = (lrow[pl.ds(16 * k, 16)]
                                          + rrow[pl.ds(16 * k, 16)])
                                    lr = jnp.maximum(sv, NEG * sv)
                                    av = av + attk[k] * lr
                                vs.append(av)
                            for sh in (8, 4, 2, 1):
                                m = (lane % (2 * sh)) < sh
                                half = len(vs) // 2
                                vs = [jnp.where(m, vs[i2], vs[i2 + half][lane ^ sh])
                                      + jnp.where(m, vs[i2][lane ^ sh],
                                                  vs[i2 + half])
                                      for i2 in range(half)]
                            sex[pl.ds(pl.multiple_of(g * 16, 16), 16)] = (
                                jnp.exp(vs[0]))
                            return 0

                        lax.fori_loop(0, CK // 16, pg, 0)

                        @pl.when(i + 1 < nck)
                        def _():
                            wait_idx(nxt)
                            issue_gath2(nxt)

                        dens = (den, den1, den2, den3)

                        def dbg(g, _):
                            dv = sdidx[pl.ds(g * 16, 16)] - nbase
                            for jj in range(16):
                                j = g * 16 + jj
                                dl = dv[jj]
                                dq = dens[jj % 4]
                                c1 = sex[pl.ds(j, 1)]
                                dq[pl.ds(dl, 1)] = dq[pl.ds(dl, 1)] + c1
                                c = c1[0]
                                arow = acc.at[dl]
                                lrow = sxl.at[j]
                                for k in range(5):
                                    sl = pl.ds(16 * k, 16)
                                    arow[sl] = arow[sl] + c * lrow[sl]
                            return 0

                        lax.fori_loop(0, ce >> 4, dbg, 0)

                        def db(j, _):
                            dl = _sget(sdidx, j) - nbase
                            c1 = sex[pl.ds(j, 1)]
                            den[pl.ds(dl, 1)] = den[pl.ds(dl, 1)] + c1
                            c = c1[0]
                            arow = acc.at[dl]
                            lrow = sxl.at[j]
                            for k in range(5):
                                sl = pl.ds(16 * k, 16)
                                arow[sl] = arow[sl] + c * lrow[sl]
                            return 0

                        lax.fori_loop((ce >> 4) << 4, ce, db, 0)

                        @pl.when(i + 2 < nck)
                        def _():
                            issue_idx(par, coff(i + 2))
                return 0

            lax.fori_loop(0, (nck + 1) >> 1, p1pair, 0)

            one = jnp.full((16,), 1.0, jnp.float32)
            for kk in range(BN // 16):
                sl = pl.ds(16 * kk, 16)
                den[sl] = one / (den[sl] + den1[sl] + den2[sl] + den3[sl]
                                 + 1e-16)

            def nrow(r, _):
                c = _sget(den, r)
                arow = acc.at[r]
                for k in range(5):
                    sl = pl.ds(16 * k, 16)
                    arow[sl] = arow[sl] * c
                return 0

            lax.fori_loop(0, BN, nrow, 0)
            pltpu.sync_copy(acc, o_hbm.at[pl.ds(nbase, BN)])

        return 0

    lax.fori_loop(0, TMAX, bucket_body, 0)


_main = pl.kernel(
    _main_body,
    compiler_params=pltpu.CompilerParams(use_tc_tiling_on_sc=False),
    out_type=jax.ShapeDtypeStruct((NB * BN, DP), jnp.float32),
    mesh=_mesh,
    name="sc_main",
    scratch_types=[
        pltpu.VMEM((NBP,), jnp.int32),
        pltpu.VMEM((NBP,), jnp.int32),
        pltpu.VMEM((NBP,), jnp.int32),
        pltpu.VMEM((NBP,), jnp.int32),
        pltpu.VMEM((DP,), jnp.float32),
        pltpu.VMEM((CK,), jnp.int32),
        pltpu.VMEM((CK,), jnp.int32),
        pltpu.VMEM((CK,), jnp.int32),
        pltpu.VMEM((CK,), jnp.int32),
        pltpu.VMEM((CK, DP), jnp.float32),
        pltpu.VMEM((CK, DP), jnp.float32),
        pltpu.VMEM((CK, DP), jnp.float32),
        pltpu.VMEM((CK, DP), jnp.float32),
        pltpu.VMEM((BN, DP), jnp.float32),
        pltpu.VMEM((BN,), jnp.float32),
        pltpu.VMEM((BN,), jnp.float32),
        pltpu.VMEM((BN,), jnp.float32),
        pltpu.VMEM((BN,), jnp.float32),
        pltpu.VMEM((CK,), jnp.float32),
        pltpu.VMEM((CK,), jnp.float32),
        pltpu.VMEM((CK,), jnp.float32),
        pltpu.SemaphoreType.DMA,
        pltpu.SemaphoreType.DMA,
        pltpu.SemaphoreType.DMA,
        pltpu.SemaphoreType.DMA,
        pltpu.SemaphoreType.DMA,
        pltpu.SemaphoreType.DMA,
    ],
)


def _conv_sc(xl, xr, att80, ei):
    srcp = jnp.pad(ei[0], (0, CK))
    dstp = jnp.pad(ei[1], (0, CK))
    hist = _hist(dstp)
    srcs, dsts, a0, c0, a1, c1 = _scat(srcp, dstp, hist)
    o = _main(xl, xr, att80, srcs, dsts, a0, c0, a1, c1)
    return o


def _fin_body(of_ref, ob_ref, xf_ref, xb_ref, bf, bb, out_ref):
    f = jnp.maximum(of_ref[:, :D] + bf[:, :D] + xf_ref[...], 0.0)
    g = jnp.maximum(ob_ref[:, :D] + bb[:, :D] + xb_ref[...], 0.0)
    out_ref[...] = jnp.concatenate([f, g], axis=-1)


def _final(of, ob, xf, xb, bf, bb):
    blk = 2000
    grid = N // blk
    ospec = pl.BlockSpec((blk, DP), lambda i: (i, 0))
    xspec = pl.BlockSpec((blk, D), lambda i: (i, 0))
    bspec = pl.BlockSpec((1, DP), lambda i: (0, 0))
    return pl.pallas_call(
        _fin_body,
        grid=grid,
        in_specs=[ospec, ospec, xspec, xspec, bspec, bspec],
        out_specs=pl.BlockSpec((blk, 2 * D), lambda i: (i, 0)),
        out_shape=jax.ShapeDtypeStruct((N, 2 * D), jnp.float32),
    )(of, ob, xf, xb, bf, bb)


def kernel(x_fwd, edge_index_fwd, x_bwd, edge_index_bwd,
           Wl_f, bl_f, Wr_f, br_f, att_f, bias_f,
           Wl_b, bl_b, Wr_b, br_b, att_b, bias_b):
    padw = lambda m: jnp.pad(m, ((0, 0), (0, DP - D)))
    padv = lambda v: jnp.pad(v, (0, DP - D)).reshape(1, DP)
    xlf, xrf, xlb, xrb = _linear(
        x_fwd, x_bwd,
        padw(Wl_f), padv(bl_f), padw(Wr_f), padv(br_f),
        padw(Wl_b), padv(bl_b), padw(Wr_b), padv(br_b))
    of = _conv_sc(xlf, xrf, jnp.pad(att_f, (0, DP - D)), edge_index_fwd)
    ob = _conv_sc(xlb, xrb, jnp.pad(att_b, (0, DP - D)), edge_index_bwd)
    return _final(of, ob, x_fwd, x_bwd, padv(bias_f), padv(bias_b))


# speedup vs baseline: 10.6823x; 1.0141x over previous
import jax
import jax.numpy as jnp
from jax import lax
from jax.experimental import pallas as pl
from jax.experimental.pallas import tpu as pltpu
from jax.experimental.pallas import tpu_sc as plsc

N = 50000
D = 73
DP = 80
E = 800000
NEG = 0.2
BSH = 8
BN = 256
NB = (N + BN - 1) // BN
NBP = 256
NW = 32
EW = E // NW
CK = 128
NCK_W = (EW + CK - 1) // CK
SPCAP = E // 2 + CK * NB + CK
EPAD = 2 * SPCAP + CK
TMAX = (NB + NW - 1) // NW

_mesh = plsc.VectorSubcoreMesh(core_axis_name="c", subcore_axis_name="s")


def _wid():
    return lax.axis_index("s") * 2 + lax.axis_index("c")


def _sget(ref, i):
    return ref[pl.ds(i, 1)][0]


def _lin_body(xf_ref, xb_ref, wlf, blf, wrf, brf, wlb, blb, wrb, brb,
              xlf_ref, xrf_ref, xlb_ref, xrb_ref):
    xf = xf_ref[...]
    xb = xb_ref[...]
    xlf_ref[...] = jnp.dot(xf, wlf[...], preferred_element_type=jnp.float32) + blf[...]
    xrf_ref[...] = jnp.dot(xf, wrf[...], preferred_element_type=jnp.float32) + brf[...]
    xlb_ref[...] = jnp.dot(xb, wlb[...], preferred_element_type=jnp.float32) + blb[...]
    xrb_ref[...] = jnp.dot(xb, wrb[...], preferred_element_type=jnp.float32) + brb[...]


def _linear(xf, xb, wlf, blf, wrf, brf, wlb, blb, wrb, brb):
    blk = 2000
    grid = N // blk
    xspec = pl.BlockSpec((blk, D), lambda i: (i, 0))
    wspec = pl.BlockSpec((D, DP), lambda i: (0, 0))
    bspec = pl.BlockSpec((1, DP), lambda i: (0, 0))
    ospec = pl.BlockSpec((blk, DP), lambda i: (i, 0))
    return pl.pallas_call(
        _lin_body,
        grid=grid,
        in_specs=[xspec, xspec, wspec, bspec, wspec, bspec,
                  wspec, bspec, wspec, bspec],
        out_specs=[ospec, ospec, ospec, ospec],
        out_shape=[jax.ShapeDtypeStruct((N, DP), jnp.float32)] * 4,
    )(xf, xb, wlf, blf, wrf, brf, wlb, blb, wrb, brb)


def _hist_body(dst_hbm, hist_hbm, dbuf, hv, hs):
    w = _wid()
    pltpu.sync_copy(dst_hbm.at[pl.ds(pl.multiple_of(w * EW, 8), EW)], dbuf)

    def zb(b, _):
        hs[b] = 0
        return 0

    lax.fori_loop(0, NBP, zb, 0)

    def egroup(g, _):
        dv = dbuf[pl.ds(g * 16, 16)] >> BSH
        for jj in range(16):
            b = dv[jj]
            hs[b] = hs[b] + 1
        return 0

    lax.fori_loop(0, EW >> 4, egroup, 0)

    def body(e, _):
        b = _sget(dbuf, e) >> BSH
        hs[b] = hs[b] + 1
        return 0

    lax.fori_loop((EW >> 4) << 4, EW, body, 0)

    def cp(b, _):
        hv[pl.ds(b, 1)] = jnp.reshape(hs[b], (1,))
        return 0

    lax.fori_loop(0, NBP, cp, 0)
    pltpu.sync_copy(hv, hist_hbm.at[w])


_hist = pl.kernel(
    _hist_body,
    out_type=jax.ShapeDtypeStruct((NW, NBP), jnp.int32),
    mesh=_mesh,
    name="sc_hist",
    scratch_types=[
        pltpu.VMEM((EW,), jnp.int32),
        pltpu.VMEM((NBP,), jnp.int32),
        pltpu.SMEM((NBP,), jnp.int32),
    ],
)


def _scat_body(src_hbm, dst_hbm, hist_hbm, srcs_hbm, dsts_hbm,
               a0_hbm, c0_hbm, a1_hbm, c1_hbm,
               hall, cur, a0_v, c0_v, a1_v, c1_v,
               sbuf, dbuf, posb, sbuf1, dbuf1, posb1, sbuf2, dbuf2, posb2,
               srcs_sp, dsts_sp, curs,
               semr, semw, semr1, semw1, semr2, semw2):
    w = _wid()
    h = w & 1
    sid = lax.axis_index("s")
    pltpu.sync_copy(hist_hbm, hall)
    zi = jnp.zeros((16,), jnp.int32)

    for kk in range(NBP // 16):
        sl = pl.ds(16 * kk, 16)

        def ws0(k, c):
            return c + hall[2 * k, sl]

        def ws1(k, c):
            return c + hall[2 * k + 1, sl]

        def wsm(k, c):
            return c + hall[2 * k + h, sl]

        c0_v[sl] = lax.fori_loop(0, 16, ws0, zi)
        c1_v[sl] = lax.fori_loop(0, 16, ws1, zi)
        cur[sl] = lax.fori_loop(0, sid, wsm, zi)

    def bloop0(b, a):
        cs = _sget(c0_v, b)
        a0_v[pl.ds(b, 1)] = jnp.reshape(a, (1,))
        return (a + cs + CK - 1) & (-CK)

    lax.fori_loop(0, NB, bloop0, 0)

    def bloop1(b, a):
        cs = _sget(c1_v, b)
        a1_v[pl.ds(b, 1)] = jnp.reshape(a, (1,))
        return (a + cs + CK - 1) & (-CK)

    lax.fori_loop(0, NB, bloop1, 0)

    def cinit(b, _):
        ab = jnp.where(h == 0, _sget(a0_v, b), _sget(a1_v, b))
        curs[b] = ab + _sget(cur, b)
        return 0

    lax.fori_loop(0, NB, cinit, 0)

    base = w * EW
    bufs = ((sbuf, dbuf, posb, semr, semw),
            (sbuf1, dbuf1, posb1, semr1, semw1),
            (sbuf2, dbuf2, posb2, semr2, semw2))

    def issue_read(p, i):
        off = pl.multiple_of(base + i * CK, 8)
        s = bufs[p]
        pltpu.async_copy(src_hbm.at[pl.ds(off, CK)], s[0], s[3])
        pltpu.async_copy(dst_hbm.at[pl.ds(off, CK)], s[1], s[3])

    def wait_read(p):
        s = bufs[p]
        pltpu.make_async_copy(src_hbm.at[pl.ds(0, CK)], s[0], s[3]).wait()
        pltpu.make_async_copy(dst_hbm.at[pl.ds(0, CK)], s[1], s[3]).wait()

    def issue_scat(p):
        s = bufs[p]
        pltpu.async_copy(s[0], srcs_sp.at[s[2]], s[4])
        pltpu.async_copy(s[1], dsts_sp.at[s[2]], s[4])

    def wait_scat(p):
        s = bufs[p]
        pltpu.make_async_copy(s[0], srcs_sp.at[s[2]], s[4]).wait()
        pltpu.make_async_copy(s[1], dsts_sp.at[s[2]], s[4]).wait()

    issue_read(0, 0)

    def chunk3(i3, _):
        for sub in range(3):
            i = i3 * 3 + sub
            par = sub
            nxt = (sub + 1) % 3

            @pl.when(i < NCK_W)
            def _():
                @pl.when(i + 1 < NCK_W)
                def _():
                    @pl.when(i >= 2)
                    def _():
                        wait_scat(nxt)
                    issue_read(nxt, i + 1)
                wait_read(par)
                s = bufs[par]
                sdbuf, sposb = s[1], s[2]
                ce = jnp.minimum(CK, EW - i * CK)

                def egroup(g, _):
                    dv = sdbuf[pl.ds(g * 16, 16)] >> BSH
                    for jj in range(16):
                        b = dv[jj]
                        p = curs[b]
                        curs[b] = p + 1
                        sposb[pl.ds(g * 16 + jj, 1)] = jnp.reshape(p, (1,))
                    return 0

                lax.fori_loop(0, ce >> 4, egroup, 0)

                def ebody(j, _):
                    b = _sget(sdbuf, j) >> BSH
                    p = curs[b]
                    curs[b] = p + 1
                    sposb[pl.ds(j, 1)] = jnp.reshape(p, (1,))
                    return 0

                lax.fori_loop((ce >> 4) << 4, ce, ebody, 0)

                def tbody(j, _):
                    sposb[pl.ds(j, 1)] = jnp.reshape(SPCAP + j, (1,))
                    return 0

                lax.fori_loop(ce, CK, tbody, 0)
                issue_scat(par)
        return 0

    lax.fori_loop(0, (NCK_W + 2) // 3, chunk3, 0)
    for p in range(3):
        wait_scat(p)
    plsc.subcore_barrier()

    @pl.when(sid == 0)
    def _():
        off = pl.multiple_of(h * SPCAP, 8)
        pltpu.async_copy(srcs_sp.at[pl.ds(0, SPCAP)],
                         srcs_hbm.at[pl.ds(off, SPCAP)], semr)
        pltpu.async_copy(dsts_sp.at[pl.ds(0, SPCAP)],
                         dsts_hbm.at[pl.ds(off, SPCAP)], semr)
        pltpu.make_async_copy(srcs_sp.at[pl.ds(0, SPCAP)],
                              srcs_hbm.at[pl.ds(off, SPCAP)], semr).wait()
        pltpu.make_async_copy(dsts_sp.at[pl.ds(0, SPCAP)],
                              dsts_hbm.at[pl.ds(off, SPCAP)], semr).wait()

    @pl.when(w == 0)
    def _():
        pltpu.sync_copy(a0_v, a0_hbm)
        pltpu.sync_copy(c0_v, c0_hbm)
        pltpu.sync_copy(a1_v, a1_hbm)
        pltpu.sync_copy(c1_v, c1_hbm)


_scat = pl.kernel(
    _scat_body,
    out_type=(
        jax.ShapeDtypeStruct((EPAD,), jnp.int32),
        jax.ShapeDtypeStruct((EPAD,), jnp.int32),
        jax.ShapeDtypeStruct((NBP,), jnp.int32),
        jax.ShapeDtypeStruct((NBP,), jnp.int32),
        jax.ShapeDtypeStruct((NBP,), jnp.int32),
        jax.ShapeDtypeStruct((NBP,), jnp.int32),
    ),
    mesh=_mesh,
    name="sc_scat",
    scratch_types=[
        pltpu.VMEM((NW, NBP), jnp.int32),
        pltpu.VMEM((NBP,), jnp.int32),
        pltpu.VMEM((NBP,), jnp.int32),
        pltpu.VMEM((NBP,), jnp.int32),
        pltpu.VMEM((NBP,), jnp.int32),
        pltpu.VMEM((NBP,), jnp.int32),
        pltpu.VMEM((CK,), jnp.int32),
        pltpu.VMEM((CK,), jnp.int32),
        pltpu.VMEM((CK,), jnp.int32),
        pltpu.VMEM((CK,), jnp.int32),
        pltpu.VMEM((CK,), jnp.int32),
        pltpu.VMEM((CK,), jnp.int32),
        pltpu.VMEM((CK,), jnp.int32),
        pltpu.VMEM((CK,), jnp.int32),
        pltpu.VMEM((CK,), jnp.int32),
        pltpu.VMEM_SHARED((SPCAP + CK,), jnp.int32),
        pltpu.VMEM_SHARED((SPCAP + CK,), jnp.int32),
        pltpu.SMEM((NBP,), jnp.int32),
        pltpu.SemaphoreType.DMA,
        pltpu.SemaphoreType.DMA,
        pltpu.SemaphoreType.DMA,
        pltpu.SemaphoreType.DMA,
        pltpu.SemaphoreType.DMA,
        pltpu.SemaphoreType.DMA,
    ],
)


def _main_body(xl_hbm, xr_hbm, att_hbm, srcs_hbm, dsts_hbm, a0_hbm,
               c0_hbm, a1_hbm, c1_hbm, o_hbm,
               s0_v, n0_v, s1_v, n1_v, attv,
               sidx0, didx0, sidx1, didx1, xlrow0, xrbuf, xlrow1, xrrow1,
               acc, den, den1, den2, den3, albuf, exbuf0, exbuf1,
               semi0, semi1, semg0, semg1, semx0, semx1):
    w = _wid()
    pltpu.sync_copy(a0_hbm, s0_v)
    pltpu.sync_copy(c0_hbm, n0_v)
    pltpu.sync_copy(a1_hbm, s1_v)
    pltpu.sync_copy(c1_hbm, n1_v)
    pltpu.sync_copy(att_hbm, attv)
    attk = [attv[pl.ds(16 * k, 16)] for k in range(5)]
    zf = jnp.zeros((16,), jnp.float32)
    zi = jnp.zeros((16,), jnp.int32)
    bufs = ((sidx0, didx0, xlrow0, None, semi0, semg0, exbuf0, semx0),
            (sidx1, didx1, xlrow1, None, semi1, semg1, exbuf1, semx1))

    def clamp_idx(ref, hi):
        for kk in range(CK // 16):
            v = ref[pl.ds(16 * kk, 16)]
            ref[pl.ds(16 * kk, 16)] = jnp.minimum(jnp.maximum(v, zi), hi)

    def issue_idx(p, off):
        s = bufs[p]
        pltpu.async_copy(srcs_hbm.at[pl.ds(off, CK)], s[0], s[4])
        pltpu.async_copy(dsts_hbm.at[pl.ds(off, CK)], s[1], s[4])

    def wait_idx(p):
        s = bufs[p]
        pltpu.make_async_copy(srcs_hbm.at[pl.ds(0, CK)], s[0], s[4]).wait()
        pltpu.make_async_copy(dsts_hbm.at[pl.ds(0, CK)], s[1], s[4]).wait()

    def issue_gath2(p):
        s = bufs[p]
        clamp_idx(s[0], N - 1)
        pltpu.async_copy(xl_hbm.at[s[0]], s[2], s[5])

    def wait_gath2(p):
        s = bufs[p]
        pltpu.make_async_copy(xl_hbm.at[s[0]], s[2], s[5]).wait()

    def bucket_body(t, _):
        b = w + NW * t

        @pl.when(b < NB)
        def _():
            st0 = _sget(s0_v, b)
            n0 = _sget(n0_v, b)
            st1 = _sget(s1_v, b) + SPCAP
            n1 = _sget(n1_v, b)
            nck0 = (n0 + CK - 1) >> 7
            nck = nck0 + ((n1 + CK - 1) >> 7)
            nbase = b * BN

            def cce(i):
                return jnp.minimum(CK, jnp.where(i < nck0, n0 - i * CK,
                                                 n1 - (i - nck0) * CK))

            def zacc(r, _):
                row = acc.at[r]
                for k in range(5):
                    row[pl.ds(16 * k, 16)] = zf
                return 0

            lax.fori_loop(0, BN, zacc, 0)
            pltpu.sync_copy(xr_hbm.at[pl.ds(nbase, BN)], xrbuf)
            for kk in range(BN // 16):
                den[pl.ds(16 * kk, 16)] = zf
                den1[pl.ds(16 * kk, 16)] = zf
                den2[pl.ds(16 * kk, 16)] = zf
                den3[pl.ds(16 * kk, 16)] = zf

            def coff(i):
                return pl.multiple_of(
                    jnp.where(i < nck0, st0 + i * CK,
                              st1 + (i - nck0) * CK), CK)

            @pl.when(nck > 0)
            def _():
                issue_idx(0, coff(0))
                wait_idx(0)
                issue_gath2(0)

                @pl.when(nck > 1)
                def _():
                    issue_idx(1, coff(1))

            def p1pair(i2, _):
                for sub in range(2):
                    i = i2 * 2 + sub
                    par = sub
                    nxt = 1 - sub
                    s = bufs[par]

                    @pl.when(i < nck)
                    def _():
                        wait_gath2(par)
                        ce = cce(i)
                        sdidx, sxl, sex = s[1], s[2], s[6]
                        lane = jnp.arange(16)

                        def pg(g, _):
                            dvc = sdidx[pl.ds(g * 16, 16)] - nbase
                            dvc = jnp.minimum(jnp.maximum(dvc, zi), BN - 1)
                            vs = []
                            for jj in range(16):
                                lrow = sxl.at[g * 16 + jj]
                                rrow = xrbuf.at[dvc[jj]]
                                av = zf
                                for k in range(5):
                                    sv = (lrow[pl.ds(16 * k, 16)]
                                          + rrow[pl.ds(16 * k, 16)])
                                    lr = jnp.maximum(sv, NEG * sv)
                                    av = av + attk[k] * lr
                                vs.append(av)
                            for sh in (8, 4, 2, 1):
                                m = (lane % (2 * sh)) < sh
                                half = len(vs) // 2
                                vs = [jnp.where(m, vs[i2], vs[i2 + half][lane ^ sh])
                                      + jnp.where(m, vs[i2][lane ^ sh],
                                                  vs[i2 + half])
                                      for i2 in range(half)]
                            sex[pl.ds(pl.multiple_of(g * 16, 16), 16)] = (
                                jnp.exp(vs[0]))
                            return 0

                        lax.fori_loop(0, CK // 16, pg, 0)

                        @pl.when(i + 1 < nck)
                        def _():
                            wait_idx(nxt)
                            issue_gath2(nxt)

                        dens = (den, den1, den2, den3)

                        def dbg(g, _):
                            dv = sdidx[pl.ds(g * 16, 16)] - nbase
                            for jj in range(16):
                                j = g * 16 + jj
                                dl = dv[jj]
                                dq = dens[jj % 4]
                                c1 = sex[pl.ds(j, 1)]
                                dq[pl.ds(dl, 1)] = dq[pl.ds(dl, 1)] + c1
                                c = c1[0]
                                arow = acc.at[dl]
                                lrow = sxl.at[j]
                                for k in range(5):
                                    sl = pl.ds(16 * k, 16)
                                    arow[sl] = arow[sl] + c * lrow[sl]
                            return 0

                        lax.fori_loop(0, ce >> 4, dbg, 0)

                        def db(j, _):
                            dl = _sget(sdidx, j) - nbase
                            c1 = sex[pl.ds(j, 1)]
                            den[pl.ds(dl, 1)] = den[pl.ds(dl, 1)] + c1
                            c = c1[0]
                            arow = acc.at[dl]
                            lrow = sxl.at[j]
                            for k in range(5):
                                sl = pl.ds(16 * k, 16)
                                arow[sl] = arow[sl] + c * lrow[sl]
                            return 0

                        lax.fori_loop((ce >> 4) << 4, ce, db, 0)

                        @pl.when(i + 2 < nck)
                        def _():
                            issue_idx(par, coff(i + 2))
                return 0

            lax.fori_loop(0, (nck + 1) >> 1, p1pair, 0)

            one = jnp.full((16,), 1.0, jnp.float32)
            for kk in range(BN // 16):
                sl = pl.ds(16 * kk, 16)
                den[sl] = one / (den[sl] + den1[sl] + den2[sl] + den3[sl]
                                 + 1e-16)

            def nrow(r, _):
                c = _sget(den, r)
                arow = acc.at[r]
                for k in range(5):
                    sl = pl.ds(16 * k, 16)
                    arow[sl] = arow[sl] * c
                return 0

            lax.fori_loop(0, BN, nrow, 0)
            pltpu.sync_copy(acc, o_hbm.at[pl.ds(nbase, BN)])

        return 0

    lax.fori_loop(0, TMAX, bucket_body, 0)


_main = pl.kernel(
    _main_body,
    compiler_params=pltpu.CompilerParams(use_tc_tiling_on_sc=False),
    out_type=jax.ShapeDtypeStruct((NB * BN, DP), jnp.float32),
    mesh=_mesh,
    name="sc_main",
    scratch_types=[
        pltpu.VMEM((NBP,), jnp.int32),
        pltpu.VMEM((NBP,), jnp.int32),
        pltpu.VMEM((NBP,), jnp.int32),
        pltpu.VMEM((NBP,), jnp.int32),
        pltpu.VMEM((DP,), jnp.float32),
        pltpu.VMEM((CK,), jnp.int32),
        pltpu.VMEM((CK,), jnp.int32),
        pltpu.VMEM((CK,), jnp.int32),
        pltpu.VMEM((CK,), jnp.int32),
        pltpu.VMEM((CK, DP), jnp.float32),
        pltpu.VMEM((BN, DP), jnp.float32),
        pltpu.VMEM((CK, DP), jnp.float32),
        pltpu.VMEM((CK, DP), jnp.float32),
        pltpu.VMEM((BN, DP), jnp.float32),
        pltpu.VMEM((BN,), jnp.float32),
        pltpu.VMEM((BN,), jnp.float32),
        pltpu.VMEM((BN,), jnp.float32),
        pltpu.VMEM((BN,), jnp.float32),
        pltpu.VMEM((CK,), jnp.float32),
        pltpu.VMEM((CK,), jnp.float32),
        pltpu.VMEM((CK,), jnp.float32),
        pltpu.SemaphoreType.DMA,
        pltpu.SemaphoreType.DMA,
        pltpu.SemaphoreType.DMA,
        pltpu.SemaphoreType.DMA,
        pltpu.SemaphoreType.DMA,
        pltpu.SemaphoreType.DMA,
    ],
)


def _conv_sc(xl, xr, att80, ei):
    srcp = jnp.pad(ei[0], (0, CK))
    dstp = jnp.pad(ei[1], (0, CK))
    hist = _hist(dstp)
    srcs, dsts, a0, c0, a1, c1 = _scat(srcp, dstp, hist)
    o = _main(xl, xr, att80, srcs, dsts, a0, c0, a1, c1)
    return o


def _fin_body(of_ref, ob_ref, xf_ref, xb_ref, bf, bb, out_ref):
    f = jnp.maximum(of_ref[:, :D] + bf[:, :D] + xf_ref[...], 0.0)
    g = jnp.maximum(ob_ref[:, :D] + bb[:, :D] + xb_ref[...], 0.0)
    out_ref[...] = jnp.concatenate([f, g], axis=-1)


def _final(of, ob, xf, xb, bf, bb):
    blk = 2000
    grid = N // blk
    ospec = pl.BlockSpec((blk, DP), lambda i: (i, 0))
    xspec = pl.BlockSpec((blk, D), lambda i: (i, 0))
    bspec = pl.BlockSpec((1, DP), lambda i: (0, 0))
    return pl.pallas_call(
        _fin_body,
        grid=grid,
        in_specs=[ospec, ospec, xspec, xspec, bspec, bspec],
        out_specs=pl.BlockSpec((blk, 2 * D), lambda i: (i, 0)),
        out_shape=jax.ShapeDtypeStruct((N, 2 * D), jnp.float32),
    )(of, ob, xf, xb, bf, bb)


def kernel(x_fwd, edge_index_fwd, x_bwd, edge_index_bwd,
           Wl_f, bl_f, Wr_f, br_f, att_f, bias_f,
           Wl_b, bl_b, Wr_b, br_b, att_b, bias_b):
    padw = lambda m: jnp.pad(m, ((0, 0), (0, DP - D)))
    padv = lambda v: jnp.pad(v, (0, DP - D)).reshape(1, DP)
    xlf, xrf, xlb, xrb = _linear(
        x_fwd, x_bwd,
        padw(Wl_f), padv(bl_f), padw(Wr_f), padv(br_f),
        padw(Wl_b), padv(bl_b), padw(Wr_b), padv(br_b))
    of = _conv_sc(xlf, xrf, jnp.pad(att_f, (0, DP - D)), edge_index_fwd)
    ob = _conv_sc(xlb, xrb, jnp.pad(att_b, (0, DP - D)), edge_index_bwd)
    return _final(of, ob, x_fwd, x_bwd, padv(bias_f), padv(bias_b))
